# baseline (speedup 1.0000x reference)
import jax
import jax.numpy as jnp
from jax.experimental import pallas as pl
from jax.experimental.pallas import tpu as pltpu

F32 = jnp.float32
N_MELS, R, E, NFFT, K = 80, 5, 256, 2048, 8
HALF = E // 2
N, TYR, TX = 16, 160, 128
TY = TYR * R
CNT = float(N * TY)


def _cp(**kw):
    return pltpu.CompilerParams(**kw)


def _prep(gru_ws, whhf, whhb, bhhf, bhhb, bank_ws, bank_gs, bank_bes):
    n_gru = len(gru_ws)
    H3 = 3 * HALF

    def body(*refs):
        ins = refs[:n_gru + 4 + K * 3]
        outs = refs[n_gru + 4 + K * 3:]
        for i in range(n_gru):
            outs[i][...] = jnp.transpose(ins[i][...])
        whhf_ref, whhb_ref, bhf_ref, bhb_ref = ins[n_gru:n_gru + 4]
        w2_ref = outs[n_gru]
        bc_ref = outs[n_gru + 1]
        w2_ref[...] = jnp.zeros_like(w2_ref)
        w2_ref[0:HALF, 0:H3] = jnp.transpose(whhf_ref[...])
        w2_ref[HALF:E, H3:2 * H3] = jnp.transpose(whhb_ref[...])
        bc_ref[:, 0:H3] = bhf_ref[...]
        bc_ref[:, H3:2 * H3] = bhb_ref[...]
        wbig_ref = outs[n_gru + 2]
        g_ref = outs[n_gru + 3]
        be_ref = outs[n_gru + 4]
        wbig_ref[...] = jnp.zeros_like(wbig_ref)
        base = n_gru + 4
        for j in range(K):
            kk = j + 1
            lo = 3 - (kk - 1) // 2
            wbig_ref[lo:lo + kk, :, j * HALF:(j + 1) * HALF] = ins[base + j][...]
            g_ref[:, j * HALF:(j + 1) * HALF] = ins[base + K + j][...]
            be_ref[:, j * HALF:(j + 1) * HALF] = ins[base + 2 * K + j][...]

    full = lambda shape: pl.BlockSpec(shape, lambda: tuple(0 for _ in shape))
    args = list(gru_ws) + [whhf, whhb, bhhf.reshape(1, H3), bhhb.reshape(1, H3)] \
        + list(bank_ws) + [g.reshape(1, HALF) for g in bank_gs] \
        + [b.reshape(1, HALF) for b in bank_bes]
    in_specs = [full(a.shape) for a in args]
    out_shapes = [jax.ShapeDtypeStruct((w.shape[1], w.shape[0]), F32)
                  for w in gru_ws]
    out_shapes += [jax.ShapeDtypeStruct((E, 2 * H3), F32),
                   jax.ShapeDtypeStruct((1, 2 * H3), F32)]
    out_shapes += [jax.ShapeDtypeStruct((8, N_MELS, K * HALF), F32),
                   jax.ShapeDtypeStruct((1, K * HALF), F32),
                   jax.ShapeDtypeStruct((1, K * HALF), F32)]
    out_specs = tuple(full(s.shape) for s in out_shapes)
    return pl.pallas_call(
        body,
        in_specs=in_specs,
        out_specs=out_specs,
        out_shape=tuple(out_shapes),
        compiler_params=_cp(vmem_limit_bytes=100 * 1024 * 1024),
        name="weight_prep",
    )(*args)


def _dense_chain(x, layers, row_block, name):
    M, K0 = x.shape
    grid = (M // row_block,)

    def body(*refs):
        x_ref = refs[0]
        o_ref = refs[-1]
        h = x_ref[...]
        for i in range(len(layers)):
            w_ref = refs[1 + 2 * i]
            b_ref = refs[2 + 2 * i]
            h = jnp.dot(h, w_ref[...], preferred_element_type=F32) + b_ref[...]
            if layers[i][2] == "relu":
                h = jnp.maximum(h, 0.0)
        o_ref[...] = h

    in_specs = [pl.BlockSpec((row_block, K0), lambda i: (i, 0))]
    args = [x]
    for (w, b, _act) in layers:
        kk, nn = w.shape
        in_specs.append(pl.BlockSpec((kk, nn), lambda i: (0, 0)))
        in_specs.append(pl.BlockSpec((1, nn), lambda i: (0, 0)))
        args.append(w)
        args.append(b.reshape(1, nn))
    n_out = layers[-1][0].shape[1]
    return pl.pallas_call(
        body,
        grid=grid,
        in_specs=in_specs,
        out_specs=pl.BlockSpec((row_block, n_out), lambda i: (i, 0)),
        out_shape=jax.ShapeDtypeStruct((M, n_out), F32),
        compiler_params=_cp(dimension_semantics=("arbitrary",)),
        name=name,
    )(*args)


def _gru_cell(xt, h, whhT_ref, bhh_ref, H):
    hw = jnp.dot(h, whhT_ref[...], preferred_element_type=F32) + bhh_ref[...]
    r = jax.nn.sigmoid(xt[:, :H] + hw[:, :H])
    z = jax.nn.sigmoid(xt[:, H:2 * H] + hw[:, H:2 * H])
    c = jnp.tanh(xt[:, 2 * H:] + r * hw[:, 2 * H:])
    return (1.0 - z) * c + z * h


def _gru_scan(xw, whhT, bhh, res=None, tb=8, name="gru_scan"):
    Nb, T, H3 = xw.shape
    H = H3 // 3
    has_res = res is not None

    def body(*refs):
        if has_res:
            xw_ref, whh_ref, bhh_ref, res_ref, ys_ref, h_ref = refs
        else:
            xw_ref, whh_ref, bhh_ref, ys_ref, h_ref = refs

        @pl.when(pl.program_id(0) == 0)
        def _():
            h_ref[...] = jnp.zeros_like(h_ref)

        h = h_ref[...]
        for j in range(tb):
            h = _gru_cell(xw_ref[:, j, :], h, whh_ref, bhh_ref, H)
            if has_res:
                ys_ref[:, j, :] = h + res_ref[:, j, :]
            else:
                ys_ref[:, j, :] = h
        h_ref[...] = h

    in_specs = [
        pl.BlockSpec((Nb, tb, H3), lambda t: (0, t, 0)),
        pl.BlockSpec((H, H3), lambda t: (0, 0)),
        pl.BlockSpec((1, H3), lambda t: (0, 0)),
    ]
    args = [xw, whhT, bhh.reshape(1, H3)]
    if has_res:
        in_specs.append(pl.BlockSpec((Nb, tb, H), lambda t: (0, t, 0)))
        args.append(res)
    return pl.pallas_call(
        body,
        grid=(T // tb,),
        in_specs=in_specs,
        out_specs=pl.BlockSpec((Nb, tb, H), lambda t: (0, t, 0)),
        out_shape=jax.ShapeDtypeStruct((Nb, T, H), F32),
        scratch_shapes=[pltpu.VMEM((Nb, H), F32)],
        compiler_params=_cp(dimension_semantics=("arbitrary",)),
        name=name,
    )(*args)


def _attention(memory, ao, W, U, v, pw, pb, wih1T, bih1, tb=32):
    Nb, T, Eh = ao.shape
    H3 = wih1T.shape[1]

    def body(mem_ref, ao_ref, w_ref, u_ref, v_ref, pwt_ref, pwb_ref, pb_ref,
             wih_ref, bih_ref, attn_ref, proj_ref, xw_ref):
        mem = mem_ref[0]
        wm = jnp.dot(mem, w_ref[...], preferred_element_type=F32)
        aob = ao_ref[0]
        uo = jnp.dot(aob, u_ref[...], preferred_element_type=F32)
        tnh = jnp.tanh(wm[None, :, :] + uo[:, None, :])
        scores = jnp.sum(tnh * v_ref[...][None, :, :], axis=-1)
        m = jnp.max(scores, axis=-1, keepdims=True)
        e = jnp.exp(scores - m)
        a = e / jnp.sum(e, axis=-1, keepdims=True)
        attn_ref[0] = a
        ctx = jnp.dot(a, mem, preferred_element_type=F32)
        proj = (jnp.dot(ctx, pwt_ref[...], preferred_element_type=F32)
                + jnp.dot(aob, pwb_ref[...], preferred_element_type=F32)
                + pb_ref[...])
        proj_ref[0] = proj
        xw_ref[0] = jnp.dot(proj, wih_ref[...], preferred_element_type=F32) + bih_ref[...]

    grid = (Nb, T // tb)
    n0 = lambda n, t: (n, 0, 0)
    nt = lambda n, t: (n, t, 0)
    full2 = lambda shape: pl.BlockSpec(shape, lambda n, t: (0, 0))
    out_shapes = (
        jax.ShapeDtypeStruct((Nb, T, TX), F32),
        jax.ShapeDtypeStruct((Nb, T, Eh), F32),
        jax.ShapeDtypeStruct((Nb, T, H3), F32),
    )
    return pl.pallas_call(
        body,
        grid=grid,
        in_specs=[
            pl.BlockSpec((1, TX, Eh), n0),
            pl.BlockSpec((1, tb, Eh), nt),
            full2((Eh, Eh)),
            full2((Eh, Eh)),
            full2((1, Eh)),
            full2((Eh, Eh)),
            full2((Eh, Eh)),
            full2((1, Eh)),
            full2((Eh, H3)),
            full2((1, H3)),
        ],
        out_specs=(
            pl.BlockSpec((1, tb, TX), nt),
            pl.BlockSpec((1, tb, Eh), nt),
            pl.BlockSpec((1, tb, H3), nt),
        ),
        out_shape=out_shapes,
        compiler_params=_cp(dimension_semantics=("arbitrary", "arbitrary"),
                            vmem_limit_bytes=100 * 1024 * 1024),
        name="attention_fused",
    )(memory, ao, W, U, v.reshape(1, Eh), pw[:Eh], pw[Eh:], pb.reshape(1, Eh),
      wih1T, bih1.reshape(1, H3))


def _conv_bank(y, wbig):
    C = wbig.shape[2]

    def body(y_ref, w_ref, bank_ref, s_ref, q_ref):
        x = y_ref[0]
        xp = jnp.concatenate(
            [jnp.zeros((3, N_MELS), F32), x, jnp.zeros((4, N_MELS), F32)], axis=0)
        acc = jnp.dot(xp[0:TY], w_ref[0], preferred_element_type=F32)
        for d in range(1, 8):
            acc += jnp.dot(xp[d:d + TY], w_ref[d], preferred_element_type=F32)
        bank_ref[0] = acc
        s_ref[0] = jnp.sum(acc, axis=0, keepdims=True)
        q_ref[0] = jnp.sum(acc * acc, axis=0, keepdims=True)

    return pl.pallas_call(
        body,
        grid=(N,),
        in_specs=[
            pl.BlockSpec((1, TY, N_MELS), lambda n: (n, 0, 0)),
            pl.BlockSpec((8, N_MELS, C), lambda n: (0, 0, 0)),
        ],
        out_specs=(
            pl.BlockSpec((1, TY, C), lambda n: (n, 0, 0)),
            pl.BlockSpec((1, 1, C), lambda n: (n, 0, 0)),
            pl.BlockSpec((1, 1, C), lambda n: (n, 0, 0)),
        ),
        out_shape=(
            jax.ShapeDtypeStruct((N, TY, C), F32),
            jax.ShapeDtypeStruct((N, 1, C), F32),
            jax.ShapeDtypeStruct((N, 1, C), F32),
        ),
        compiler_params=_cp(dimension_semantics=("arbitrary",),
                            vmem_limit_bytes=100 * 1024 * 1024),
        name="conv_bank",
    )(y, wbig)


def _bn_scale_shift(s_ref, q_ref, g_ref, be_ref):
    m = jnp.sum(s_ref[...], axis=0) / CNT
    var = jnp.sum(q_ref[...], axis=0) / CNT - m * m
    scale = g_ref[...] * jax.lax.rsqrt(var + 1e-5)
    shift = be_ref[...] - m * scale
    return scale, shift


def _conv_proj(xraw, ssum, ssq, g, be, w3, relu_in, pool_in, name):
    Cin = xraw.shape[2]
    Cout = w3.shape[2]

    def body(x_ref, s_in_ref, q_in_ref, g_ref, be_ref, w_ref,
             o_ref, s_ref, q_ref):
        scale, shift = _bn_scale_shift(s_in_ref, q_in_ref, g_ref, be_ref)
        a = x_ref[0] * scale + shift
        if relu_in:
            a = jnp.maximum(a, 0.0)
        if pool_in:
            an = jnp.concatenate([a[1:], jnp.zeros((1, Cin), F32)], axis=0)
            a = jnp.maximum(a, an)
        zp = jnp.zeros((1, Cin), F32)
        prv = jnp.concatenate([zp, a[:-1]], axis=0)
        nxt = jnp.concatenate([a[1:], zp], axis=0)
        acc = (jnp.dot(prv, w_ref[0], preferred_element_type=F32)
               + jnp.dot(a, w_ref[1], preferred_element_type=F32)
               + jnp.dot(nxt, w_ref[2], preferred_element_type=F32))
        o_ref[0] = acc
        s_ref[0] = jnp.sum(acc, axis=0, keepdims=True)
        q_ref[0] = jnp.sum(acc * acc, axis=0, keepdims=True)

    return pl.pallas_call(
        body,
        grid=(N,),
        in_specs=[
            pl.BlockSpec((1, TY, Cin), lambda n: (n, 0, 0)),
            pl.BlockSpec((N, 1, Cin), lambda n: (0, 0, 0)),
            pl.BlockSpec((N, 1, Cin), lambda n: (0, 0, 0)),
            pl.BlockSpec((1, Cin), lambda n: (0, 0)),
            pl.BlockSpec((1, Cin), lambda n: (0, 0)),
            pl.BlockSpec((3, Cin, Cout), lambda n: (0, 0, 0)),
        ],
        out_specs=(
            pl.BlockSpec((1, TY, Cout), lambda n: (n, 0, 0)),
            pl.BlockSpec((1, 1, Cout), lambda n: (n, 0, 0)),
            pl.BlockSpec((1, 1, Cout), lambda n: (n, 0, 0)),
        ),
        out_shape=(
            jax.ShapeDtypeStruct((N, TY, Cout), F32),
            jax.ShapeDtypeStruct((N, 1, Cout), F32),
            jax.ShapeDtypeStruct((N, 1, Cout), F32),
        ),
        compiler_params=_cp(dimension_semantics=("arbitrary",),
                            vmem_limit_bytes=100 * 1024 * 1024),
        name=name,
    )(xraw, ssum.reshape(N, 1, Cin), ssq.reshape(N, 1, Cin),
      g.reshape(1, Cin), be.reshape(1, Cin), w3)


def _highway(c2f, yf, ssum, ssq, g, be, wpre, bpre, hw_ws, wfT, bf, wbT, bb,
             row_block=512):
    M = c2f.shape[0]
    H3 = wfT.shape[1]

    def body(*refs):
        (c2_ref, y_ref, s_ref, q_ref, g_ref, be_ref, wp_ref, bp_ref) = refs[:8]
        hw_refs = refs[8:8 + 16]
        (wf_ref, bf_ref, wb_ref, bb_ref, xf_ref, xb_ref) = refs[24:]
        scale, shift = _bn_scale_shift(s_ref, q_ref, g_ref, be_ref)
        h0 = c2_ref[...] * scale + shift + y_ref[...]
        h = jnp.dot(h0, wp_ref[...], preferred_element_type=F32) + bp_ref[...]
        for i in range(4):
            w1_ref, b1_ref, w2_ref, b2_ref = hw_refs[4 * i:4 * i + 4]
            hh = jnp.maximum(
                jnp.dot(h, w1_ref[...], preferred_element_type=F32) + b1_ref[...], 0.0)
            tt = jax.nn.sigmoid(
                jnp.dot(h, w2_ref[...], preferred_element_type=F32) + b2_ref[...])
            h = hh * tt + h * (1.0 - tt)
        xf_ref[...] = jnp.dot(h, wf_ref[...], preferred_element_type=F32) + bf_ref[...]
        xb_ref[...] = jnp.dot(h, wb_ref[...], preferred_element_type=F32) + bb_ref[...]

    full = lambda shape: pl.BlockSpec(shape, lambda i: tuple(0 for _ in shape))
    rows = lambda c: pl.BlockSpec((row_block, c), lambda i: (i, 0))
    hw_args, hw_specs = [], []
    for (w1, b1, w2, b2) in hw_ws:
        hw_args += [w1, b1.reshape(1, HALF), w2, b2.reshape(1, HALF)]
        hw_specs += [full((HALF, HALF)), full((1, HALF)),
                     full((HALF, HALF)), full((1, HALF))]
    return pl.pallas_call(
        body,
        grid=(M // row_block,),
        in_specs=[
            rows(N_MELS), rows(N_MELS),
            full((N, 1, N_MELS)), full((N, 1, N_MELS)),
            full((1, N_MELS)), full((1, N_MELS)),
            full((N_MELS, HALF)), full((1, HALF)),
        ] + hw_specs + [
            full((HALF, H3)), full((1, H3)),
            full((HALF, H3)), full((1, H3)),
        ],
        out_specs=(
            pl.BlockSpec((row_block, H3), lambda i: (i, 0)),
            pl.BlockSpec((row_block, H3), lambda i: (i, 0)),
        ),
        out_shape=(
            jax.ShapeDtypeStruct((M, H3), F32),
            jax.ShapeDtypeStruct((M, H3), F32),
        ),
        compiler_params=_cp(dimension_semantics=("arbitrary",)),
        name="highway_stack",
    )(c2f, yf, ssum.reshape(N, 1, N_MELS), ssq.reshape(N, 1, N_MELS),
      g.reshape(1, N_MELS), be.reshape(1, N_MELS),
      wpre, bpre.reshape(1, HALF), *hw_args,
      wfT, bf.reshape(1, H3), wbT, bb.reshape(1, H3))


def _gru_gates(xt, hw, h, H):
    r = jax.nn.sigmoid(xt[:, :H] + hw[:, :H])
    z = jax.nn.sigmoid(xt[:, H:2 * H] + hw[:, H:2 * H])
    c = jnp.tanh(xt[:, 2 * H:] + r * hw[:, 2 * H:])
    return (1.0 - z) * c + z * h


def _bigru(xwf, xwb, w2cat, bhh_cat, tb=8):
    Nb, T, H3 = xwf.shape
    H = H3 // 3
    nblk = T // tb

    def body(xf_ref, xb_ref, w_ref, b_ref, yf_ref, yb_ref, h_ref):
        @pl.when(pl.program_id(0) == 0)
        def _():
            h_ref[...] = jnp.zeros_like(h_ref)

        hcat = h_ref[...]
        for j in range(tb):
            hw = jnp.dot(hcat, w_ref[...], preferred_element_type=F32) + b_ref[...]
            jj = tb - 1 - j
            hf = _gru_gates(xf_ref[:, j, :], hw[:, :H3], hcat[:, :H], H)
            hb = _gru_gates(xb_ref[:, jj, :], hw[:, H3:], hcat[:, H:], H)
            yf_ref[:, j, :] = hf
            yb_ref[:, jj, :] = hb
            hcat = jnp.concatenate([hf, hb], axis=1)
        h_ref[...] = hcat

    fwd = lambda t: (0, t, 0)
    rev = lambda t: (0, nblk - 1 - t, 0)
    return pl.pallas_call(
        body,
        grid=(nblk,),
        in_specs=[
            pl.BlockSpec((Nb, tb, H3), fwd),
            pl.BlockSpec((Nb, tb, H3), rev),
            pl.BlockSpec((2 * H, 2 * H3), lambda t: (0, 0)),
            pl.BlockSpec((1, 2 * H3), lambda t: (0, 0)),
        ],
        out_specs=(
            pl.BlockSpec((Nb, tb, H), fwd),
            pl.BlockSpec((Nb, tb, H), rev),
        ),
        out_shape=(
            jax.ShapeDtypeStruct((Nb, T, H), F32),
            jax.ShapeDtypeStruct((Nb, T, H), F32),
        ),
        scratch_shapes=[pltpu.VMEM((Nb, 2 * H), F32)],
        compiler_params=_cp(dimension_semantics=("arbitrary",)),
        name="bigru",
    )(xwf, xwb, w2cat, bhh_cat)


def _fc2(f, b, w, bias, row_block=512):
    M = f.shape[0]
    Hh = f.shape[1]
    n_out = w.shape[1]

    def body(f_ref, b_ref, w_ref, bi_ref, o_ref):
        o = (jnp.dot(f_ref[...], w_ref[:Hh], preferred_element_type=F32)
             + jnp.dot(b_ref[...], w_ref[Hh:], preferred_element_type=F32)
             + bi_ref[...])
        o_ref[...] = jnp.transpose(o)

    return pl.pallas_call(
        body,
        grid=(M // row_block,),
        in_specs=[
            pl.BlockSpec((row_block, Hh), lambda i: (i, 0)),
            pl.BlockSpec((row_block, Hh), lambda i: (i, 0)),
            pl.BlockSpec((2 * Hh, n_out), lambda i: (0, 0)),
            pl.BlockSpec((1, n_out), lambda i: (0, 0)),
        ],
        out_specs=pl.BlockSpec((n_out, row_block), lambda i: (0, i)),
        out_shape=jax.ShapeDtypeStruct((n_out, M), F32),
        compiler_params=_cp(dimension_semantics=("arbitrary",)),
        name="fc2",
    )(f, b, w, bias.reshape(1, n_out))


def kernel(inputs, memory, params):
    p = params
    NT = N * TYR
    NTY = N * TY

    (wa_ihT, wa_hhT, g1_ihT, g1_hhT, g2_ihT, g2_hhT,
     bgf_ihT, bgb_ihT, w2cat, bhh_cat, wbig, g_cat, be_cat) = _prep(
        [p['attn_gru']['wih'], p['attn_gru']['whh'],
         p['gru1']['wih'], p['gru1']['whh'],
         p['gru2']['wih'], p['gru2']['whh'],
         p['bigru_f']['wih'], p['bigru_b']['wih']],
        p['bigru_f']['whh'], p['bigru_b']['whh'],
        p['bigru_f']['bhh'], p['bigru_b']['bhh'],
        [c['w'] for c in p['bank']],
        [c['g'] for c in p['bank']],
        [c['be'] for c in p['bank']])

    xw_a = _dense_chain(
        inputs.reshape(NT, N_MELS),
        [(p['prenet']['w1'], p['prenet']['b1'], "relu"),
         (p['prenet']['w2'], p['prenet']['b2'], "relu"),
         (wa_ihT, p['attn_gru']['bih'], "none")],
        row_block=256, name="prenet_xw")
    ao = _gru_scan(xw_a.reshape(N, TYR, 3 * E), wa_hhT,
                   p['attn_gru']['bhh'], name="attn_gru")

    attn, proj, xw1 = _attention(
        memory, ao, p['attn']['W'], p['attn']['U'], p['attn']['v'],
        p['proj']['w'], p['proj']['b'], g1_ihT, p['gru1']['bih'])

    g1 = _gru_scan(xw1, g1_hhT, p['gru1']['bhh'], res=proj, name="gru1")
    xw2 = _dense_chain(g1.reshape(NT, E),
                       [(g2_ihT, p['gru2']['bih'], "none")],
                       row_block=256, name="gru2_xw")
    g2 = _gru_scan(xw2.reshape(N, TYR, 3 * E), g2_hhT,
                   p['gru2']['bhh'], res=g1, name="gru2")

    mels = _dense_chain(g2.reshape(NT, E),
                        [(p['fc1']['w'], p['fc1']['b'], "none")],
                        row_block=256, name="fc1").reshape(N, TYR, N_MELS * R)
    y = mels.reshape(N, TY, N_MELS)

    bank, bs, bq = _conv_bank(y, wbig)
    c1, c1s, c1q = _conv_proj(bank, bs, bq, g_cat, be_cat, p['proj1']['w'],
                              relu_in=True, pool_in=True, name="proj1_conv")
    c2, c2s, c2q = _conv_proj(c1, c1s, c1q, p['proj1']['g'], p['proj1']['be'],
                              p['proj2']['w'],
                              relu_in=True, pool_in=False, name="proj2_conv")

    hw_ws = [(hw['w1'], hw['b1'], hw['w2'], hw['b2']) for hw in p['hw']]
    xwf, xwb = _highway(
        c2.reshape(NTY, N_MELS), y.reshape(NTY, N_MELS),
        c2s, c2q, p['proj2']['g'], p['proj2']['be'],
        p['pre_hw']['w'], p['pre_hw']['b'], hw_ws,
        bgf_ihT, p['bigru_f']['bih'], bgb_ihT, p['bigru_b']['bih'])

    f, bwd = _bigru(xwf.reshape(N, TY, 3 * HALF), xwb.reshape(N, TY, 3 * HALF),
                    w2cat, bhh_cat)

    mags_t = _fc2(f.reshape(NTY, HALF), bwd.reshape(NTY, HALF),
                  p['fc2']['w'], p['fc2']['b'])
    mags = mags_t.reshape(1 + NFFT // 2, N, TY).transpose(1, 2, 0)
    return mels, mags, attn


# speedup vs baseline: 1.0129x; 1.0129x over previous
import jax
import jax.numpy as jnp
from jax.experimental import pallas as pl
from jax.experimental.pallas import tpu as pltpu

F32 = jnp.float32
N_MELS, R, E, NFFT, K = 80, 5, 256, 2048, 8
HALF = E // 2
N, TYR, TX = 16, 160, 128
TY = TYR * R
CNT = float(N * TY)


def _cp(**kw):
    return pltpu.CompilerParams(**kw)


def _prep(gru_ws, bank_ws, bank_gs, bank_bes):
    n_gru = len(gru_ws)

    def body(*refs):
        ins = refs[:n_gru + K * 3]
        outs = refs[n_gru + K * 3:]
        for i in range(n_gru):
            outs[i][...] = jnp.transpose(ins[i][...])
        wbig_ref = outs[n_gru]
        g_ref = outs[n_gru + 1]
        be_ref = outs[n_gru + 2]
        wbig_ref[...] = jnp.zeros_like(wbig_ref)
        for j in range(K):
            kk = j + 1
            lo = 3 - (kk - 1) // 2
            wbig_ref[lo:lo + kk, :, j * HALF:(j + 1) * HALF] = ins[n_gru + j][...]
            g_ref[:, j * HALF:(j + 1) * HALF] = ins[n_gru + K + j][...]
            be_ref[:, j * HALF:(j + 1) * HALF] = ins[n_gru + 2 * K + j][...]

    full = lambda shape: pl.BlockSpec(shape, lambda: tuple(0 for _ in shape))
    args = list(gru_ws) + list(bank_ws) + \
        [g.reshape(1, HALF) for g in bank_gs] + \
        [b.reshape(1, HALF) for b in bank_bes]
    in_specs = [full(a.shape) for a in args]
    out_shapes = [jax.ShapeDtypeStruct((w.shape[1], w.shape[0]), F32)
                  for w in gru_ws]
    out_shapes += [jax.ShapeDtypeStruct((8, N_MELS, K * HALF), F32),
                   jax.ShapeDtypeStruct((1, K * HALF), F32),
                   jax.ShapeDtypeStruct((1, K * HALF), F32)]
    out_specs = tuple(full(s.shape) for s in out_shapes)
    return pl.pallas_call(
        body,
        in_specs=in_specs,
        out_specs=out_specs,
        out_shape=tuple(out_shapes),
        compiler_params=_cp(vmem_limit_bytes=100 * 1024 * 1024),
        name="weight_prep",
    )(*args)


def _dense_chain(x, layers, row_block, name):
    M, K0 = x.shape
    grid = (M // row_block,)

    def body(*refs):
        x_ref = refs[0]
        o_ref = refs[-1]
        h = x_ref[...]
        for i in range(len(layers)):
            w_ref = refs[1 + 2 * i]
            b_ref = refs[2 + 2 * i]
            h = jnp.dot(h, w_ref[...], preferred_element_type=F32) + b_ref[...]
            if layers[i][2] == "relu":
                h = jnp.maximum(h, 0.0)
        o_ref[...] = h

    in_specs = [pl.BlockSpec((row_block, K0), lambda i: (i, 0))]
    args = [x]
    for (w, b, _act) in layers:
        kk, nn = w.shape
        in_specs.append(pl.BlockSpec((kk, nn), lambda i: (0, 0)))
        in_specs.append(pl.BlockSpec((1, nn), lambda i: (0, 0)))
        args.append(w)
        args.append(b.reshape(1, nn))
    n_out = layers[-1][0].shape[1]
    return pl.pallas_call(
        body,
        grid=grid,
        in_specs=in_specs,
        out_specs=pl.BlockSpec((row_block, n_out), lambda i: (i, 0)),
        out_shape=jax.ShapeDtypeStruct((M, n_out), F32),
        compiler_params=_cp(dimension_semantics=("arbitrary",)),
        name=name,
    )(*args)


def _gru_cell(xt, h, whhT_ref, bhh_ref, H):
    hw = jnp.dot(h, whhT_ref[...], preferred_element_type=F32) + bhh_ref[...]
    r = jax.nn.sigmoid(xt[:, :H] + hw[:, :H])
    z = jax.nn.sigmoid(xt[:, H:2 * H] + hw[:, H:2 * H])
    c = jnp.tanh(xt[:, 2 * H:] + r * hw[:, 2 * H:])
    return (1.0 - z) * c + z * h


def _gru_scan(xw, whhT, bhh, res=None, tb=8, name="gru_scan"):
    Nb, T, H3 = xw.shape
    H = H3 // 3
    has_res = res is not None

    def body(*refs):
        if has_res:
            xw_ref, whh_ref, bhh_ref, res_ref, ys_ref, h_ref = refs
        else:
            xw_ref, whh_ref, bhh_ref, ys_ref, h_ref = refs

        @pl.when(pl.program_id(0) == 0)
        def _():
            h_ref[...] = jnp.zeros_like(h_ref)

        h = h_ref[...]
        for j in range(tb):
            h = _gru_cell(xw_ref[:, j, :], h, whh_ref, bhh_ref, H)
            if has_res:
                ys_ref[:, j, :] = h + res_ref[:, j, :]
            else:
                ys_ref[:, j, :] = h
        h_ref[...] = h

    in_specs = [
        pl.BlockSpec((Nb, tb, H3), lambda t: (0, t, 0)),
        pl.BlockSpec((H, H3), lambda t: (0, 0)),
        pl.BlockSpec((1, H3), lambda t: (0, 0)),
    ]
    args = [xw, whhT, bhh.reshape(1, H3)]
    if has_res:
        in_specs.append(pl.BlockSpec((Nb, tb, H), lambda t: (0, t, 0)))
        args.append(res)
    return pl.pallas_call(
        body,
        grid=(T // tb,),
        in_specs=in_specs,
        out_specs=pl.BlockSpec((Nb, tb, H), lambda t: (0, t, 0)),
        out_shape=jax.ShapeDtypeStruct((Nb, T, H), F32),
        scratch_shapes=[pltpu.VMEM((Nb, H), F32)],
        compiler_params=_cp(dimension_semantics=("arbitrary",)),
        name=name,
    )(*args)


def _attention(memory, ao, W, U, v, pw, pb, wih1T, bih1, tb=32):
    Nb, T, Eh = ao.shape
    H3 = wih1T.shape[1]

    def body(mem_ref, ao_ref, w_ref, u_ref, v_ref, pwt_ref, pwb_ref, pb_ref,
             wih_ref, bih_ref, attn_ref, proj_ref, xw_ref):
        mem = mem_ref[0]
        wm = jnp.dot(mem, w_ref[...], preferred_element_type=F32)
        aob = ao_ref[0]
        uo = jnp.dot(aob, u_ref[...], preferred_element_type=F32)
        tnh = jnp.tanh(wm[None, :, :] + uo[:, None, :])
        scores = jnp.sum(tnh * v_ref[...][None, :, :], axis=-1)
        m = jnp.max(scores, axis=-1, keepdims=True)
        e = jnp.exp(scores - m)
        a = e / jnp.sum(e, axis=-1, keepdims=True)
        attn_ref[0] = a
        ctx = jnp.dot(a, mem, preferred_element_type=F32)
        proj = (jnp.dot(ctx, pwt_ref[...], preferred_element_type=F32)
                + jnp.dot(aob, pwb_ref[...], preferred_element_type=F32)
                + pb_ref[...])
        proj_ref[0] = proj
        xw_ref[0] = jnp.dot(proj, wih_ref[...], preferred_element_type=F32) + bih_ref[...]

    grid = (Nb, T // tb)
    n0 = lambda n, t: (n, 0, 0)
    nt = lambda n, t: (n, t, 0)
    full2 = lambda shape: pl.BlockSpec(shape, lambda n, t: (0, 0))
    out_shapes = (
        jax.ShapeDtypeStruct((Nb, T, TX), F32),
        jax.ShapeDtypeStruct((Nb, T, Eh), F32),
        jax.ShapeDtypeStruct((Nb, T, H3), F32),
    )
    return pl.pallas_call(
        body,
        grid=grid,
        in_specs=[
            pl.BlockSpec((1, TX, Eh), n0),
            pl.BlockSpec((1, tb, Eh), nt),
            full2((Eh, Eh)),
            full2((Eh, Eh)),
            full2((1, Eh)),
            full2((Eh, Eh)),
            full2((Eh, Eh)),
            full2((1, Eh)),
            full2((Eh, H3)),
            full2((1, H3)),
        ],
        out_specs=(
            pl.BlockSpec((1, tb, TX), nt),
            pl.BlockSpec((1, tb, Eh), nt),
            pl.BlockSpec((1, tb, H3), nt),
        ),
        out_shape=out_shapes,
        compiler_params=_cp(dimension_semantics=("arbitrary", "arbitrary"),
                            vmem_limit_bytes=100 * 1024 * 1024),
        name="attention_fused",
    )(memory, ao, W, U, v.reshape(1, Eh), pw[:Eh], pw[Eh:], pb.reshape(1, Eh),
      wih1T, bih1.reshape(1, H3))


def _conv_bank(y, wbig):
    C = wbig.shape[2]

    def body(y_ref, w_ref, bank_ref, s_ref, q_ref):
        x = y_ref[0]
        xp = jnp.concatenate(
            [jnp.zeros((3, N_MELS), F32), x, jnp.zeros((4, N_MELS), F32)], axis=0)
        acc = jnp.dot(xp[0:TY], w_ref[0], preferred_element_type=F32)
        for d in range(1, 8):
            acc += jnp.dot(xp[d:d + TY], w_ref[d], preferred_element_type=F32)
        bank_ref[0] = acc
        s_ref[0] = jnp.sum(acc, axis=0, keepdims=True)
        q_ref[0] = jnp.sum(acc * acc, axis=0, keepdims=True)

    return pl.pallas_call(
        body,
        grid=(N,),
        in_specs=[
            pl.BlockSpec((1, TY, N_MELS), lambda n: (n, 0, 0)),
            pl.BlockSpec((8, N_MELS, C), lambda n: (0, 0, 0)),
        ],
        out_specs=(
            pl.BlockSpec((1, TY, C), lambda n: (n, 0, 0)),
            pl.BlockSpec((1, 1, C), lambda n: (n, 0, 0)),
            pl.BlockSpec((1, 1, C), lambda n: (n, 0, 0)),
        ),
        out_shape=(
            jax.ShapeDtypeStruct((N, TY, C), F32),
            jax.ShapeDtypeStruct((N, 1, C), F32),
            jax.ShapeDtypeStruct((N, 1, C), F32),
        ),
        compiler_params=_cp(dimension_semantics=("arbitrary",),
                            vmem_limit_bytes=100 * 1024 * 1024),
        name="conv_bank",
    )(y, wbig)


def _bn_scale_shift(s_ref, q_ref, g_ref, be_ref):
    m = jnp.sum(s_ref[...], axis=0) / CNT
    var = jnp.sum(q_ref[...], axis=0) / CNT - m * m
    scale = g_ref[...] * jax.lax.rsqrt(var + 1e-5)
    shift = be_ref[...] - m * scale
    return scale, shift


def _conv_proj(xraw, ssum, ssq, g, be, w3, relu_in, pool_in, name):
    Cin = xraw.shape[2]
    Cout = w3.shape[2]

    def body(x_ref, s_in_ref, q_in_ref, g_ref, be_ref, w_ref,
             o_ref, s_ref, q_ref):
        scale, shift = _bn_scale_shift(s_in_ref, q_in_ref, g_ref, be_ref)
        a = x_ref[0] * scale + shift
        if relu_in:
            a = jnp.maximum(a, 0.0)
        if pool_in:
            an = jnp.concatenate([a[1:], jnp.zeros((1, Cin), F32)], axis=0)
            a = jnp.maximum(a, an)
        zp = jnp.zeros((1, Cin), F32)
        prv = jnp.concatenate([zp, a[:-1]], axis=0)
        nxt = jnp.concatenate([a[1:], zp], axis=0)
        acc = (jnp.dot(prv, w_ref[0], preferred_element_type=F32)
               + jnp.dot(a, w_ref[1], preferred_element_type=F32)
               + jnp.dot(nxt, w_ref[2], preferred_element_type=F32))
        o_ref[0] = acc
        s_ref[0] = jnp.sum(acc, axis=0, keepdims=True)
        q_ref[0] = jnp.sum(acc * acc, axis=0, keepdims=True)

    return pl.pallas_call(
        body,
        grid=(N,),
        in_specs=[
            pl.BlockSpec((1, TY, Cin), lambda n: (n, 0, 0)),
            pl.BlockSpec((N, 1, Cin), lambda n: (0, 0, 0)),
            pl.BlockSpec((N, 1, Cin), lambda n: (0, 0, 0)),
            pl.BlockSpec((1, Cin), lambda n: (0, 0)),
            pl.BlockSpec((1, Cin), lambda n: (0, 0)),
            pl.BlockSpec((3, Cin, Cout), lambda n: (0, 0, 0)),
        ],
        out_specs=(
            pl.BlockSpec((1, TY, Cout), lambda n: (n, 0, 0)),
            pl.BlockSpec((1, 1, Cout), lambda n: (n, 0, 0)),
            pl.BlockSpec((1, 1, Cout), lambda n: (n, 0, 0)),
        ),
        out_shape=(
            jax.ShapeDtypeStruct((N, TY, Cout), F32),
            jax.ShapeDtypeStruct((N, 1, Cout), F32),
            jax.ShapeDtypeStruct((N, 1, Cout), F32),
        ),
        compiler_params=_cp(dimension_semantics=("arbitrary",),
                            vmem_limit_bytes=100 * 1024 * 1024),
        name=name,
    )(xraw, ssum.reshape(N, 1, Cin), ssq.reshape(N, 1, Cin),
      g.reshape(1, Cin), be.reshape(1, Cin), w3)


def _highway(c2f, yf, ssum, ssq, g, be, wpre, bpre, hw_ws, wfT, bf, wbT, bb,
             row_block=512):
    M = c2f.shape[0]
    H3 = wfT.shape[1]

    def body(*refs):
        (c2_ref, y_ref, s_ref, q_ref, g_ref, be_ref, wp_ref, bp_ref) = refs[:8]
        hw_refs = refs[8:8 + 16]
        (wf_ref, bf_ref, wb_ref, bb_ref, xf_ref, xb_ref) = refs[24:]
        scale, shift = _bn_scale_shift(s_ref, q_ref, g_ref, be_ref)
        h0 = c2_ref[...] * scale + shift + y_ref[...]
        h = jnp.dot(h0, wp_ref[...], preferred_element_type=F32) + bp_ref[...]
        for i in range(4):
            w1_ref, b1_ref, w2_ref, b2_ref = hw_refs[4 * i:4 * i + 4]
            hh = jnp.maximum(
                jnp.dot(h, w1_ref[...], preferred_element_type=F32) + b1_ref[...], 0.0)
            tt = jax.nn.sigmoid(
                jnp.dot(h, w2_ref[...], preferred_element_type=F32) + b2_ref[...])
            h = hh * tt + h * (1.0 - tt)
        xf_ref[...] = jnp.dot(h, wf_ref[...], preferred_element_type=F32) + bf_ref[...]
        xb_ref[...] = jnp.dot(h, wb_ref[...], preferred_element_type=F32) + bb_ref[...]

    full = lambda shape: pl.BlockSpec(shape, lambda i: tuple(0 for _ in shape))
    rows = lambda c: pl.BlockSpec((row_block, c), lambda i: (i, 0))
    hw_args, hw_specs = [], []
    for (w1, b1, w2, b2) in hw_ws:
        hw_args += [w1, b1.reshape(1, HALF), w2, b2.reshape(1, HALF)]
        hw_specs += [full((HALF, HALF)), full((1, HALF)),
                     full((HALF, HALF)), full((1, HALF))]
    return pl.pallas_call(
        body,
        grid=(M // row_block,),
        in_specs=[
            rows(N_MELS), rows(N_MELS),
            full((N, 1, N_MELS)), full((N, 1, N_MELS)),
            full((1, N_MELS)), full((1, N_MELS)),
            full((N_MELS, HALF)), full((1, HALF)),
        ] + hw_specs + [
            full((HALF, H3)), full((1, H3)),
            full((HALF, H3)), full((1, H3)),
        ],
        out_specs=(
            pl.BlockSpec((row_block, H3), lambda i: (i, 0)),
            pl.BlockSpec((row_block, H3), lambda i: (i, 0)),
        ),
        out_shape=(
            jax.ShapeDtypeStruct((M, H3), F32),
            jax.ShapeDtypeStruct((M, H3), F32),
        ),
        compiler_params=_cp(dimension_semantics=("arbitrary",)),
        name="highway_stack",
    )(c2f, yf, ssum.reshape(N, 1, N_MELS), ssq.reshape(N, 1, N_MELS),
      g.reshape(1, N_MELS), be.reshape(1, N_MELS),
      wpre, bpre.reshape(1, HALF), *hw_args,
      wfT, bf.reshape(1, H3), wbT, bb.reshape(1, H3))


def _bigru(xwf, xwb, whhTf, bhhf, whhTb, bhhb, tb=8):
    Nb, T, H3 = xwf.shape
    H = H3 // 3
    nblk = T // tb

    def body(xf_ref, xb_ref, wf_ref, bf_ref, wb_ref, bb_ref,
             yf_ref, yb_ref, hf_ref, hb_ref):
        @pl.when(pl.program_id(0) == 0)
        def _():
            hf_ref[...] = jnp.zeros_like(hf_ref)
            hb_ref[...] = jnp.zeros_like(hb_ref)

        hf = hf_ref[...]
        hb = hb_ref[...]
        for j in range(tb):
            hf = _gru_cell(xf_ref[:, j, :], hf, wf_ref, bf_ref, H)
            yf_ref[:, j, :] = hf
            jj = tb - 1 - j
            hb = _gru_cell(xb_ref[:, jj, :], hb, wb_ref, bb_ref, H)
            yb_ref[:, jj, :] = hb
        hf_ref[...] = hf
        hb_ref[...] = hb

    fwd = lambda t: (0, t, 0)
    rev = lambda t: (0, nblk - 1 - t, 0)
    return pl.pallas_call(
        body,
        grid=(nblk,),
        in_specs=[
            pl.BlockSpec((Nb, tb, H3), fwd),
            pl.BlockSpec((Nb, tb, H3), rev),
            pl.BlockSpec((H, H3), lambda t: (0, 0)),
            pl.BlockSpec((1, H3), lambda t: (0, 0)),
            pl.BlockSpec((H, H3), lambda t: (0, 0)),
            pl.BlockSpec((1, H3), lambda t: (0, 0)),
        ],
        out_specs=(
            pl.BlockSpec((Nb, tb, H), fwd),
            pl.BlockSpec((Nb, tb, H), rev),
        ),
        out_shape=(
            jax.ShapeDtypeStruct((Nb, T, H), F32),
            jax.ShapeDtypeStruct((Nb, T, H), F32),
        ),
        scratch_shapes=[pltpu.VMEM((Nb, H), F32), pltpu.VMEM((Nb, H), F32)],
        compiler_params=_cp(dimension_semantics=("arbitrary",)),
        name="bigru",
    )(xwf, xwb, whhTf, bhhf, whhTb, bhhb)


def _fc2(f, b, w, bias, row_block=512):
    M = f.shape[0]
    Hh = f.shape[1]
    n_out = w.shape[1]

    def body(f_ref, b_ref, w_ref, bi_ref, o_ref):
        o = (jnp.dot(f_ref[...], w_ref[:Hh], preferred_element_type=F32)
             + jnp.dot(b_ref[...], w_ref[Hh:], preferred_element_type=F32)
             + bi_ref[...])
        o_ref[...] = jnp.transpose(o)

    return pl.pallas_call(
        body,
        grid=(M // row_block,),
        in_specs=[
            pl.BlockSpec((row_block, Hh), lambda i: (i, 0)),
            pl.BlockSpec((row_block, Hh), lambda i: (i, 0)),
            pl.BlockSpec((2 * Hh, n_out), lambda i: (0, 0)),
            pl.BlockSpec((1, n_out), lambda i: (0, 0)),
        ],
        out_specs=pl.BlockSpec((n_out, row_block), lambda i: (0, i)),
        out_shape=jax.ShapeDtypeStruct((n_out, M), F32),
        compiler_params=_cp(dimension_semantics=("arbitrary",)),
        name="fc2",
    )(f, b, w, bias.reshape(1, n_out))


def kernel(inputs, memory, params):
    p = params
    NT = N * TYR
    NTY = N * TY

    (wa_ihT, wa_hhT, g1_ihT, g1_hhT, g2_ihT, g2_hhT,
     bgf_ihT, bgf_hhT, bgb_ihT, bgb_hhT, wbig, g_cat, be_cat) = _prep(
        [p['attn_gru']['wih'], p['attn_gru']['whh'],
         p['gru1']['wih'], p['gru1']['whh'],
         p['gru2']['wih'], p['gru2']['whh'],
         p['bigru_f']['wih'], p['bigru_f']['whh'],
         p['bigru_b']['wih'], p['bigru_b']['whh']],
        [c['w'] for c in p['bank']],
        [c['g'] for c in p['bank']],
        [c['be'] for c in p['bank']])

    xw_a = _dense_chain(
        inputs.reshape(NT, N_MELS),
        [(p['prenet']['w1'], p['prenet']['b1'], "relu"),
         (p['prenet']['w2'], p['prenet']['b2'], "relu"),
         (wa_ihT, p['attn_gru']['bih'], "none")],
        row_block=256, name="prenet_xw")
    ao = _gru_scan(xw_a.reshape(N, TYR, 3 * E), wa_hhT,
                   p['attn_gru']['bhh'], name="attn_gru")

    attn, proj, xw1 = _attention(
        memory, ao, p['attn']['W'], p['attn']['U'], p['attn']['v'],
        p['proj']['w'], p['proj']['b'], g1_ihT, p['gru1']['bih'])

    g1 = _gru_scan(xw1, g1_hhT, p['gru1']['bhh'], res=proj, name="gru1")
    xw2 = _dense_chain(g1.reshape(NT, E),
                       [(g2_ihT, p['gru2']['bih'], "none")],
                       row_block=256, name="gru2_xw")
    g2 = _gru_scan(xw2.reshape(N, TYR, 3 * E), g2_hhT,
                   p['gru2']['bhh'], res=g1, name="gru2")

    mels = _dense_chain(g2.reshape(NT, E),
                        [(p['fc1']['w'], p['fc1']['b'], "none")],
                        row_block=256, name="fc1").reshape(N, TYR, N_MELS * R)
    y = mels.reshape(N, TY, N_MELS)

    bank, bs, bq = _conv_bank(y, wbig)
    c1, c1s, c1q = _conv_proj(bank, bs, bq, g_cat, be_cat, p['proj1']['w'],
                              relu_in=True, pool_in=True, name="proj1_conv")
    c2, c2s, c2q = _conv_proj(c1, c1s, c1q, p['proj1']['g'], p['proj1']['be'],
                              p['proj2']['w'],
                              relu_in=True, pool_in=False, name="proj2_conv")

    hw_ws = [(hw['w1'], hw['b1'], hw['w2'], hw['b2']) for hw in p['hw']]
    xwf, xwb = _highway(
        c2.reshape(NTY, N_MELS), y.reshape(NTY, N_MELS),
        c2s, c2q, p['proj2']['g'], p['proj2']['be'],
        p['pre_hw']['w'], p['pre_hw']['b'], hw_ws,
        bgf_ihT, p['bigru_f']['bih'], bgb_ihT, p['bigru_b']['bih'])

    f, bwd = _bigru(xwf.reshape(N, TY, 3 * HALF), xwb.reshape(N, TY, 3 * HALF),
                    bgf_hhT, p['bigru_f']['bhh'].reshape(1, 3 * HALF),
                    bgb_hhT, p['bigru_b']['bhh'].reshape(1, 3 * HALF))

    mags_t = _fc2(f.reshape(NTY, HALF), bwd.reshape(NTY, HALF),
                  p['fc2']['w'], p['fc2']['b'])
    mags = mags_t.reshape(1 + NFFT // 2, N, TY).transpose(1, 2, 0)
    return mels, mags, attn


# speedup vs baseline: 1.0148x; 1.0019x over previous
import jax
import jax.numpy as jnp
from jax.experimental import pallas as pl
from jax.experimental.pallas import tpu as pltpu

F32 = jnp.float32
N_MELS, R, E, NFFT, K = 80, 5, 256, 2048, 8
HALF = E // 2
N, TYR, TX = 16, 160, 128
TY = TYR * R
CNT = float(N * TY)


def _cp(**kw):
    return pltpu.CompilerParams(**kw)


def _prep(gru_ws, bank_ws, bank_gs, bank_bes):
    n_gru = len(gru_ws)

    def body(*refs):
        ins = refs[:n_gru + K * 3]
        outs = refs[n_gru + K * 3:]
        for i in range(n_gru):
            outs[i][...] = jnp.transpose(ins[i][...])
        wbig_ref = outs[n_gru]
        g_ref = outs[n_gru + 1]
        be_ref = outs[n_gru + 2]
        wbig_ref[...] = jnp.zeros_like(wbig_ref)
        for j in range(K):
            kk = j + 1
            lo = 3 - (kk - 1) // 2
            wbig_ref[lo:lo + kk, :, j * HALF:(j + 1) * HALF] = ins[n_gru + j][...]
            g_ref[:, j * HALF:(j + 1) * HALF] = ins[n_gru + K + j][...]
            be_ref[:, j * HALF:(j + 1) * HALF] = ins[n_gru + 2 * K + j][...]

    full = lambda shape: pl.BlockSpec(shape, lambda: tuple(0 for _ in shape))
    args = list(gru_ws) + list(bank_ws) + \
        [g.reshape(1, HALF) for g in bank_gs] + \
        [b.reshape(1, HALF) for b in bank_bes]
    in_specs = [full(a.shape) for a in args]
    out_shapes = [jax.ShapeDtypeStruct((w.shape[1], w.shape[0]), F32)
                  for w in gru_ws]
    out_shapes += [jax.ShapeDtypeStruct((8, N_MELS, K * HALF), F32),
                   jax.ShapeDtypeStruct((1, K * HALF), F32),
                   jax.ShapeDtypeStruct((1, K * HALF), F32)]
    out_specs = tuple(full(s.shape) for s in out_shapes)
    return pl.pallas_call(
        body,
        in_specs=in_specs,
        out_specs=out_specs,
        out_shape=tuple(out_shapes),
        compiler_params=_cp(vmem_limit_bytes=100 * 1024 * 1024),
        name="weight_prep",
    )(*args)


def _dense_chain(x, layers, row_block, name):
    M, K0 = x.shape
    grid = (M // row_block,)

    def body(*refs):
        x_ref = refs[0]
        o_ref = refs[-1]
        h = x_ref[...]
        for i in range(len(layers)):
            w_ref = refs[1 + 2 * i]
            b_ref = refs[2 + 2 * i]
            h = jnp.dot(h, w_ref[...], preferred_element_type=F32) + b_ref[...]
            if layers[i][2] == "relu":
                h = jnp.maximum(h, 0.0)
        o_ref[...] = h

    in_specs = [pl.BlockSpec((row_block, K0), lambda i: (i, 0))]
    args = [x]
    for (w, b, _act) in layers:
        kk, nn = w.shape
        in_specs.append(pl.BlockSpec((kk, nn), lambda i: (0, 0)))
        in_specs.append(pl.BlockSpec((1, nn), lambda i: (0, 0)))
        args.append(w)
        args.append(b.reshape(1, nn))
    n_out = layers[-1][0].shape[1]
    return pl.pallas_call(
        body,
        grid=grid,
        in_specs=in_specs,
        out_specs=pl.BlockSpec((row_block, n_out), lambda i: (i, 0)),
        out_shape=jax.ShapeDtypeStruct((M, n_out), F32),
        compiler_params=_cp(dimension_semantics=("arbitrary",)),
        name=name,
    )(*args)


def _gru_cell(xt, h, whhT_ref, bhh_ref, H):
    hw = jnp.dot(h, whhT_ref[...], preferred_element_type=F32) + bhh_ref[...]
    r = jax.nn.sigmoid(xt[:, :H] + hw[:, :H])
    z = jax.nn.sigmoid(xt[:, H:2 * H] + hw[:, H:2 * H])
    c = jnp.tanh(xt[:, 2 * H:] + r * hw[:, 2 * H:])
    return (1.0 - z) * c + z * h


def _gru_scan(xw, whhT, bhh, res=None, tb=16, name="gru_scan"):
    Nb, T, H3 = xw.shape
    H = H3 // 3
    has_res = res is not None

    def body(*refs):
        if has_res:
            xw_ref, whh_ref, bhh_ref, res_ref, ys_ref, h_ref = refs
        else:
            xw_ref, whh_ref, bhh_ref, ys_ref, h_ref = refs

        @pl.when(pl.program_id(0) == 0)
        def _():
            h_ref[...] = jnp.zeros_like(h_ref)

        h = h_ref[...]
        for j in range(tb):
            h = _gru_cell(xw_ref[:, j, :], h, whh_ref, bhh_ref, H)
            if has_res:
                ys_ref[:, j, :] = h + res_ref[:, j, :]
            else:
                ys_ref[:, j, :] = h
        h_ref[...] = h

    in_specs = [
        pl.BlockSpec((Nb, tb, H3), lambda t: (0, t, 0)),
        pl.BlockSpec((H, H3), lambda t: (0, 0)),
        pl.BlockSpec((1, H3), lambda t: (0, 0)),
    ]
    args = [xw, whhT, bhh.reshape(1, H3)]
    if has_res:
        in_specs.append(pl.BlockSpec((Nb, tb, H), lambda t: (0, t, 0)))
        args.append(res)
    return pl.pallas_call(
        body,
        grid=(T // tb,),
        in_specs=in_specs,
        out_specs=pl.BlockSpec((Nb, tb, H), lambda t: (0, t, 0)),
        out_shape=jax.ShapeDtypeStruct((Nb, T, H), F32),
        scratch_shapes=[pltpu.VMEM((Nb, H), F32)],
        compiler_params=_cp(dimension_semantics=("arbitrary",)),
        name=name,
    )(*args)


def _attention(memory, ao, W, U, v, pw, pb, wih1T, bih1, tb=32):
    Nb, T, Eh = ao.shape
    H3 = wih1T.shape[1]

    def body(mem_ref, ao_ref, w_ref, u_ref, v_ref, pwt_ref, pwb_ref, pb_ref,
             wih_ref, bih_ref, attn_ref, proj_ref, xw_ref):
        mem = mem_ref[0]
        wm = jnp.dot(mem, w_ref[...], preferred_element_type=F32)
        aob = ao_ref[0]
        uo = jnp.dot(aob, u_ref[...], preferred_element_type=F32)
        tnh = jnp.tanh(wm[None, :, :] + uo[:, None, :])
        scores = jnp.sum(tnh * v_ref[...][None, :, :], axis=-1)
        m = jnp.max(scores, axis=-1, keepdims=True)
        e = jnp.exp(scores - m)
        a = e / jnp.sum(e, axis=-1, keepdims=True)
        attn_ref[0] = a
        ctx = jnp.dot(a, mem, preferred_element_type=F32)
        proj = (jnp.dot(ctx, pwt_ref[...], preferred_element_type=F32)
                + jnp.dot(aob, pwb_ref[...], preferred_element_type=F32)
                + pb_ref[...])
        proj_ref[0] = proj
        xw_ref[0] = jnp.dot(proj, wih_ref[...], preferred_element_type=F32) + bih_ref[...]

    grid = (Nb, T // tb)
    n0 = lambda n, t: (n, 0, 0)
    nt = lambda n, t: (n, t, 0)
    full2 = lambda shape: pl.BlockSpec(shape, lambda n, t: (0, 0))
    out_shapes = (
        jax.ShapeDtypeStruct((Nb, T, TX), F32),
        jax.ShapeDtypeStruct((Nb, T, Eh), F32),
        jax.ShapeDtypeStruct((Nb, T, H3), F32),
    )
    return pl.pallas_call(
        body,
        grid=grid,
        in_specs=[
            pl.BlockSpec((1, TX, Eh), n0),
            pl.BlockSpec((1, tb, Eh), nt),
            full2((Eh, Eh)),
            full2((Eh, Eh)),
            full2((1, Eh)),
            full2((Eh, Eh)),
            full2((Eh, Eh)),
            full2((1, Eh)),
            full2((Eh, H3)),
            full2((1, H3)),
        ],
        out_specs=(
            pl.BlockSpec((1, tb, TX), nt),
            pl.BlockSpec((1, tb, Eh), nt),
            pl.BlockSpec((1, tb, H3), nt),
        ),
        out_shape=out_shapes,
        compiler_params=_cp(dimension_semantics=("arbitrary", "arbitrary"),
                            vmem_limit_bytes=100 * 1024 * 1024),
        name="attention_fused",
    )(memory, ao, W, U, v.reshape(1, Eh), pw[:Eh], pw[Eh:], pb.reshape(1, Eh),
      wih1T, bih1.reshape(1, H3))


def _conv_bank(y, wbig):
    C = wbig.shape[2]

    def body(y_ref, w_ref, bank_ref, s_ref, q_ref):
        x = y_ref[0]
        xp = jnp.concatenate(
            [jnp.zeros((3, N_MELS), F32), x, jnp.zeros((4, N_MELS), F32)], axis=0)
        acc = jnp.dot(xp[0:TY], w_ref[0], preferred_element_type=F32)
        for d in range(1, 8):
            acc += jnp.dot(xp[d:d + TY], w_ref[d], preferred_element_type=F32)
        bank_ref[0] = acc
        s_ref[0] = jnp.sum(acc, axis=0, keepdims=True)
        q_ref[0] = jnp.sum(acc * acc, axis=0, keepdims=True)

    return pl.pallas_call(
        body,
        grid=(N,),
        in_specs=[
            pl.BlockSpec((1, TY, N_MELS), lambda n: (n, 0, 0)),
            pl.BlockSpec((8, N_MELS, C), lambda n: (0, 0, 0)),
        ],
        out_specs=(
            pl.BlockSpec((1, TY, C), lambda n: (n, 0, 0)),
            pl.BlockSpec((1, 1, C), lambda n: (n, 0, 0)),
            pl.BlockSpec((1, 1, C), lambda n: (n, 0, 0)),
        ),
        out_shape=(
            jax.ShapeDtypeStruct((N, TY, C), F32),
            jax.ShapeDtypeStruct((N, 1, C), F32),
            jax.ShapeDtypeStruct((N, 1, C), F32),
        ),
        compiler_params=_cp(dimension_semantics=("arbitrary",),
                            vmem_limit_bytes=100 * 1024 * 1024),
        name="conv_bank",
    )(y, wbig)


def _bn_scale_shift(s_ref, q_ref, g_ref, be_ref):
    m = jnp.sum(s_ref[...], axis=0) / CNT
    var = jnp.sum(q_ref[...], axis=0) / CNT - m * m
    scale = g_ref[...] * jax.lax.rsqrt(var + 1e-5)
    shift = be_ref[...] - m * scale
    return scale, shift


def _conv_proj(xraw, ssum, ssq, g, be, w3, relu_in, pool_in, name):
    Cin = xraw.shape[2]
    Cout = w3.shape[2]

    def body(x_ref, s_in_ref, q_in_ref, g_ref, be_ref, w_ref,
             o_ref, s_ref, q_ref):
        scale, shift = _bn_scale_shift(s_in_ref, q_in_ref, g_ref, be_ref)
        a = x_ref[0] * scale + shift
        if relu_in:
            a = jnp.maximum(a, 0.0)
        if pool_in:
            an = jnp.concatenate([a[1:], jnp.zeros((1, Cin), F32)], axis=0)
            a = jnp.maximum(a, an)
        zp = jnp.zeros((1, Cin), F32)
        prv = jnp.concatenate([zp, a[:-1]], axis=0)
        nxt = jnp.concatenate([a[1:], zp], axis=0)
        acc = (jnp.dot(prv, w_ref[0], preferred_element_type=F32)
               + jnp.dot(a, w_ref[1], preferred_element_type=F32)
               + jnp.dot(nxt, w_ref[2], preferred_element_type=F32))
        o_ref[0] = acc
        s_ref[0] = jnp.sum(acc, axis=0, keepdims=True)
        q_ref[0] = jnp.sum(acc * acc, axis=0, keepdims=True)

    return pl.pallas_call(
        body,
        grid=(N,),
        in_specs=[
            pl.BlockSpec((1, TY, Cin), lambda n: (n, 0, 0)),
            pl.BlockSpec((N, 1, Cin), lambda n: (0, 0, 0)),
            pl.BlockSpec((N, 1, Cin), lambda n: (0, 0, 0)),
            pl.BlockSpec((1, Cin), lambda n: (0, 0)),
            pl.BlockSpec((1, Cin), lambda n: (0, 0)),
            pl.BlockSpec((3, Cin, Cout), lambda n: (0, 0, 0)),
        ],
        out_specs=(
            pl.BlockSpec((1, TY, Cout), lambda n: (n, 0, 0)),
            pl.BlockSpec((1, 1, Cout), lambda n: (n, 0, 0)),
            pl.BlockSpec((1, 1, Cout), lambda n: (n, 0, 0)),
        ),
        out_shape=(
            jax.ShapeDtypeStruct((N, TY, Cout), F32),
            jax.ShapeDtypeStruct((N, 1, Cout), F32),
            jax.ShapeDtypeStruct((N, 1, Cout), F32),
        ),
        compiler_params=_cp(dimension_semantics=("arbitrary",),
                            vmem_limit_bytes=100 * 1024 * 1024),
        name=name,
    )(xraw, ssum.reshape(N, 1, Cin), ssq.reshape(N, 1, Cin),
      g.reshape(1, Cin), be.reshape(1, Cin), w3)


def _highway(c2f, yf, ssum, ssq, g, be, wpre, bpre, hw_ws, wfT, bf, wbT, bb,
             row_block=512):
    M = c2f.shape[0]
    H3 = wfT.shape[1]

    def body(*refs):
        (c2_ref, y_ref, s_ref, q_ref, g_ref, be_ref, wp_ref, bp_ref) = refs[:8]
        hw_refs = refs[8:8 + 16]
        (wf_ref, bf_ref, wb_ref, bb_ref, xf_ref, xb_ref) = refs[24:]
        scale, shift = _bn_scale_shift(s_ref, q_ref, g_ref, be_ref)
        h0 = c2_ref[...] * scale + shift + y_ref[...]
        h = jnp.dot(h0, wp_ref[...], preferred_element_type=F32) + bp_ref[...]
        for i in range(4):
            w1_ref, b1_ref, w2_ref, b2_ref = hw_refs[4 * i:4 * i + 4]
            hh = jnp.maximum(
                jnp.dot(h, w1_ref[...], preferred_element_type=F32) + b1_ref[...], 0.0)
            tt = jax.nn.sigmoid(
                jnp.dot(h, w2_ref[...], preferred_element_type=F32) + b2_ref[...])
            h = hh * tt + h * (1.0 - tt)
        xf_ref[...] = jnp.dot(h, wf_ref[...], preferred_element_type=F32) + bf_ref[...]
        xb_ref[...] = jnp.dot(h, wb_ref[...], preferred_element_type=F32) + bb_ref[...]

    full = lambda shape: pl.BlockSpec(shape, lambda i: tuple(0 for _ in shape))
    rows = lambda c: pl.BlockSpec((row_block, c), lambda i: (i, 0))
    hw_args, hw_specs = [], []
    for (w1, b1, w2, b2) in hw_ws:
        hw_args += [w1, b1.reshape(1, HALF), w2, b2.reshape(1, HALF)]
        hw_specs += [full((HALF, HALF)), full((1, HALF)),
                     full((HALF, HALF)), full((1, HALF))]
    return pl.pallas_call(
        body,
        grid=(M // row_block,),
        in_specs=[
            rows(N_MELS), rows(N_MELS),
            full((N, 1, N_MELS)), full((N, 1, N_MELS)),
            full((1, N_MELS)), full((1, N_MELS)),
            full((N_MELS, HALF)), full((1, HALF)),
        ] + hw_specs + [
            full((HALF, H3)), full((1, H3)),
            full((HALF, H3)), full((1, H3)),
        ],
        out_specs=(
            pl.BlockSpec((row_block, H3), lambda i: (i, 0)),
            pl.BlockSpec((row_block, H3), lambda i: (i, 0)),
        ),
        out_shape=(
            jax.ShapeDtypeStruct((M, H3), F32),
            jax.ShapeDtypeStruct((M, H3), F32),
        ),
        compiler_params=_cp(dimension_semantics=("arbitrary",)),
        name="highway_stack",
    )(c2f, yf, ssum.reshape(N, 1, N_MELS), ssq.reshape(N, 1, N_MELS),
      g.reshape(1, N_MELS), be.reshape(1, N_MELS),
      wpre, bpre.reshape(1, HALF), *hw_args,
      wfT, bf.reshape(1, H3), wbT, bb.reshape(1, H3))


def _bigru(xwf, xwb, whhTf, bhhf, whhTb, bhhb, tb=16):
    Nb, T, H3 = xwf.shape
    H = H3 // 3
    nblk = T // tb

    def body(xf_ref, xb_ref, wf_ref, bf_ref, wb_ref, bb_ref,
             yf_ref, yb_ref, hf_ref, hb_ref):
        @pl.when(pl.program_id(0) == 0)
        def _():
            hf_ref[...] = jnp.zeros_like(hf_ref)
            hb_ref[...] = jnp.zeros_like(hb_ref)

        hf = hf_ref[...]
        hb = hb_ref[...]
        for j in range(tb):
            hf = _gru_cell(xf_ref[:, j, :], hf, wf_ref, bf_ref, H)
            yf_ref[:, j, :] = hf
            jj = tb - 1 - j
            hb = _gru_cell(xb_ref[:, jj, :], hb, wb_ref, bb_ref, H)
            yb_ref[:, jj, :] = hb
        hf_ref[...] = hf
        hb_ref[...] = hb

    fwd = lambda t: (0, t, 0)
    rev = lambda t: (0, nblk - 1 - t, 0)
    return pl.pallas_call(
        body,
        grid=(nblk,),
        in_specs=[
            pl.BlockSpec((Nb, tb, H3), fwd),
            pl.BlockSpec((Nb, tb, H3), rev),
            pl.BlockSpec((H, H3), lambda t: (0, 0)),
            pl.BlockSpec((1, H3), lambda t: (0, 0)),
            pl.BlockSpec((H, H3), lambda t: (0, 0)),
            pl.BlockSpec((1, H3), lambda t: (0, 0)),
        ],
        out_specs=(
            pl.BlockSpec((Nb, tb, H), fwd),
            pl.BlockSpec((Nb, tb, H), rev),
        ),
        out_shape=(
            jax.ShapeDtypeStruct((Nb, T, H), F32),
            jax.ShapeDtypeStruct((Nb, T, H), F32),
        ),
        scratch_shapes=[pltpu.VMEM((Nb, H), F32), pltpu.VMEM((Nb, H), F32)],
        compiler_params=_cp(dimension_semantics=("arbitrary",)),
        name="bigru",
    )(xwf, xwb, whhTf, bhhf, whhTb, bhhb)


def _fc2(f, b, w, bias, row_block=512):
    M = f.shape[0]
    Hh = f.shape[1]
    n_out = w.shape[1]

    def body(f_ref, b_ref, w_ref, bi_ref, o_ref):
        o = (jnp.dot(f_ref[...], w_ref[:Hh], preferred_element_type=F32)
             + jnp.dot(b_ref[...], w_ref[Hh:], preferred_element_type=F32)
             + bi_ref[...])
        o_ref[...] = jnp.transpose(o)

    return pl.pallas_call(
        body,
        grid=(M // row_block,),
        in_specs=[
            pl.BlockSpec((row_block, Hh), lambda i: (i, 0)),
            pl.BlockSpec((row_block, Hh), lambda i: (i, 0)),
            pl.BlockSpec((2 * Hh, n_out), lambda i: (0, 0)),
            pl.BlockSpec((1, n_out), lambda i: (0, 0)),
        ],
        out_specs=pl.BlockSpec((n_out, row_block), lambda i: (0, i)),
        out_shape=jax.ShapeDtypeStruct((n_out, M), F32),
        compiler_params=_cp(dimension_semantics=("arbitrary",)),
        name="fc2",
    )(f, b, w, bias.reshape(1, n_out))


def kernel(inputs, memory, params):
    p = params
    NT = N * TYR
    NTY = N * TY

    (wa_ihT, wa_hhT, g1_ihT, g1_hhT, g2_ihT, g2_hhT,
     bgf_ihT, bgf_hhT, bgb_ihT, bgb_hhT, wbig, g_cat, be_cat) = _prep(
        [p['attn_gru']['wih'], p['attn_gru']['whh'],
         p['gru1']['wih'], p['gru1']['whh'],
         p['gru2']['wih'], p['gru2']['whh'],
         p['bigru_f']['wih'], p['bigru_f']['whh'],
         p['bigru_b']['wih'], p['bigru_b']['whh']],
        [c['w'] for c in p['bank']],
        [c['g'] for c in p['bank']],
        [c['be'] for c in p['bank']])

    xw_a = _dense_chain(
        inputs.reshape(NT, N_MELS),
        [(p['prenet']['w1'], p['prenet']['b1'], "relu"),
         (p['prenet']['w2'], p['prenet']['b2'], "relu"),
         (wa_ihT, p['attn_gru']['bih'], "none")],
        row_block=256, name="prenet_xw")
    ao = _gru_scan(xw_a.reshape(N, TYR, 3 * E), wa_hhT,
                   p['attn_gru']['bhh'], name="attn_gru")

    attn, proj, xw1 = _attention(
        memory, ao, p['attn']['W'], p['attn']['U'], p['attn']['v'],
        p['proj']['w'], p['proj']['b'], g1_ihT, p['gru1']['bih'])

    g1 = _gru_scan(xw1, g1_hhT, p['gru1']['bhh'], res=proj, name="gru1")
    xw2 = _dense_chain(g1.reshape(NT, E),
                       [(g2_ihT, p['gru2']['bih'], "none")],
                       row_block=256, name="gru2_xw")
    g2 = _gru_scan(xw2.reshape(N, TYR, 3 * E), g2_hhT,
                   p['gru2']['bhh'], res=g1, name="gru2")

    mels = _dense_chain(g2.reshape(NT, E),
                        [(p['fc1']['w'], p['fc1']['b'], "none")],
                        row_block=256, name="fc1").reshape(N, TYR, N_MELS * R)
    y = mels.reshape(N, TY, N_MELS)

    bank, bs, bq = _conv_bank(y, wbig)
    c1, c1s, c1q = _conv_proj(bank, bs, bq, g_cat, be_cat, p['proj1']['w'],
                              relu_in=True, pool_in=True, name="proj1_conv")
    c2, c2s, c2q = _conv_proj(c1, c1s, c1q, p['proj1']['g'], p['proj1']['be'],
                              p['proj2']['w'],
                              relu_in=True, pool_in=False, name="proj2_conv")

    hw_ws = [(hw['w1'], hw['b1'], hw['w2'], hw['b2']) for hw in p['hw']]
    xwf, xwb = _highway(
        c2.reshape(NTY, N_MELS), y.reshape(NTY, N_MELS),
        c2s, c2q, p['proj2']['g'], p['proj2']['be'],
        p['pre_hw']['w'], p['pre_hw']['b'], hw_ws,
        bgf_ihT, p['bigru_f']['bih'], bgb_ihT, p['bigru_b']['bih'])

    f, bwd = _bigru(xwf.reshape(N, TY, 3 * HALF), xwb.reshape(N, TY, 3 * HALF),
                    bgf_hhT, p['bigru_f']['bhh'].reshape(1, 3 * HALF),
                    bgb_hhT, p['bigru_b']['bhh'].reshape(1, 3 * HALF))

    mags_t = _fc2(f.reshape(NTY, HALF), bwd.reshape(NTY, HALF),
                  p['fc2']['w'], p['fc2']['b'])
    mags = mags_t.reshape(1 + NFFT // 2, N, TY).transpose(1, 2, 0)
    return mels, mags, attn


# speedup vs baseline: 1.0320x; 1.0170x over previous
import jax
import jax.numpy as jnp
from jax.experimental import pallas as pl
from jax.experimental.pallas import tpu as pltpu

F32 = jnp.float32
N_MELS, R, E, NFFT, K = 80, 5, 256, 2048, 8
HALF = E // 2
N, TYR, TX = 16, 160, 128
TY = TYR * R
CNT = float(N * TY)


def _cp(**kw):
    return pltpu.CompilerParams(**kw)


def _prep(gru_ws, bank_ws, bank_gs, bank_bes):
    n_gru = len(gru_ws)

    def body(*refs):
        ins = refs[:n_gru + K * 3]
        outs = refs[n_gru + K * 3:]
        for i in range(n_gru):
            outs[i][...] = jnp.transpose(ins[i][...])
        wbig_ref = outs[n_gru]
        g_ref = outs[n_gru + 1]
        be_ref = outs[n_gru + 2]
        wbig_ref[...] = jnp.zeros_like(wbig_ref)
        for j in range(K):
            kk = j + 1
            lo = 3 - (kk - 1) // 2
            wbig_ref[lo:lo + kk, :, j * HALF:(j + 1) * HALF] = ins[n_gru + j][...]
            g_ref[:, j * HALF:(j + 1) * HALF] = ins[n_gru + K + j][...]
            be_ref[:, j * HALF:(j + 1) * HALF] = ins[n_gru + 2 * K + j][...]

    full = lambda shape: pl.BlockSpec(shape, lambda: tuple(0 for _ in shape))
    args = list(gru_ws) + list(bank_ws) + \
        [g.reshape(1, HALF) for g in bank_gs] + \
        [b.reshape(1, HALF) for b in bank_bes]
    in_specs = [full(a.shape) for a in args]
    out_shapes = [jax.ShapeDtypeStruct((w.shape[1], w.shape[0]), F32)
                  for w in gru_ws]
    out_shapes += [jax.ShapeDtypeStruct((8, N_MELS, K * HALF), F32),
                   jax.ShapeDtypeStruct((1, K * HALF), F32),
                   jax.ShapeDtypeStruct((1, K * HALF), F32)]
    out_specs = tuple(full(s.shape) for s in out_shapes)
    return pl.pallas_call(
        body,
        in_specs=in_specs,
        out_specs=out_specs,
        out_shape=tuple(out_shapes),
        compiler_params=_cp(vmem_limit_bytes=100 * 1024 * 1024),
        name="weight_prep",
    )(*args)


def _dense_chain(x, layers, row_block, name):
    M, K0 = x.shape
    grid = (M // row_block,)

    def body(*refs):
        x_ref = refs[0]
        o_ref = refs[-1]
        h = x_ref[...]
        for i in range(len(layers)):
            w_ref = refs[1 + 2 * i]
            b_ref = refs[2 + 2 * i]
            h = jnp.dot(h, w_ref[...], preferred_element_type=F32) + b_ref[...]
            if layers[i][2] == "relu":
                h = jnp.maximum(h, 0.0)
        o_ref[...] = h

    in_specs = [pl.BlockSpec((row_block, K0), lambda i: (i, 0))]
    args = [x]
    for (w, b, _act) in layers:
        kk, nn = w.shape
        in_specs.append(pl.BlockSpec((kk, nn), lambda i: (0, 0)))
        in_specs.append(pl.BlockSpec((1, nn), lambda i: (0, 0)))
        args.append(w)
        args.append(b.reshape(1, nn))
    n_out = layers[-1][0].shape[1]
    return pl.pallas_call(
        body,
        grid=grid,
        in_specs=in_specs,
        out_specs=pl.BlockSpec((row_block, n_out), lambda i: (i, 0)),
        out_shape=jax.ShapeDtypeStruct((M, n_out), F32),
        compiler_params=_cp(dimension_semantics=("arbitrary",)),
        name=name,
    )(*args)


def _gru_cell(xt, h, whhT_ref, bhh_ref, H):
    hw = jnp.dot(h, whhT_ref[...], preferred_element_type=F32) + bhh_ref[...]
    r = jax.nn.sigmoid(xt[:, :H] + hw[:, :H])
    z = jax.nn.sigmoid(xt[:, H:2 * H] + hw[:, H:2 * H])
    c = jnp.tanh(xt[:, 2 * H:] + r * hw[:, 2 * H:])
    return (1.0 - z) * c + z * h


def _gru_scan(xw, whhT, bhh, res=None, tb=16, name="gru_scan"):
    Nb, T, H3 = xw.shape
    H = H3 // 3
    has_res = res is not None

    def body(*refs):
        if has_res:
            xw_ref, whh_ref, bhh_ref, res_ref, ys_ref, h_ref = refs
        else:
            xw_ref, whh_ref, bhh_ref, ys_ref, h_ref = refs

        @pl.when(pl.program_id(0) == 0)
        def _():
            h_ref[...] = jnp.zeros_like(h_ref)

        h = h_ref[...]
        for j in range(tb):
            h = _gru_cell(xw_ref[:, j, :], h, whh_ref, bhh_ref, H)
            if has_res:
                ys_ref[:, j, :] = h + res_ref[:, j, :]
            else:
                ys_ref[:, j, :] = h
        h_ref[...] = h

    in_specs = [
        pl.BlockSpec((Nb, tb, H3), lambda t: (0, t, 0)),
        pl.BlockSpec((H, H3), lambda t: (0, 0)),
        pl.BlockSpec((1, H3), lambda t: (0, 0)),
    ]
    args = [xw, whhT, bhh.reshape(1, H3)]
    if has_res:
        in_specs.append(pl.BlockSpec((Nb, tb, H), lambda t: (0, t, 0)))
        args.append(res)
    return pl.pallas_call(
        body,
        grid=(T // tb,),
        in_specs=in_specs,
        out_specs=pl.BlockSpec((Nb, tb, H), lambda t: (0, t, 0)),
        out_shape=jax.ShapeDtypeStruct((Nb, T, H), F32),
        scratch_shapes=[pltpu.VMEM((Nb, H), F32)],
        compiler_params=_cp(dimension_semantics=("arbitrary",)),
        name=name,
    )(*args)


def _attention(memory, ao, W, U, v, pw, pb, wih1T, bih1, tb=40):
    Nb, T, Eh = ao.shape
    H3 = wih1T.shape[1]

    def body(mem_ref, ao_ref, w_ref, u_ref, v_ref, pwt_ref, pwb_ref, pb_ref,
             wih_ref, bih_ref, attn_ref, proj_ref, xw_ref):
        mem = mem_ref[0]
        wm = jnp.dot(mem, w_ref[...], preferred_element_type=F32)
        aob = ao_ref[0]
        uo = jnp.dot(aob, u_ref[...], preferred_element_type=F32)
        tnh = jnp.tanh(wm[None, :, :] + uo[:, None, :])
        scores = jnp.sum(tnh * v_ref[...][None, :, :], axis=-1)
        m = jnp.max(scores, axis=-1, keepdims=True)
        e = jnp.exp(scores - m)
        a = e / jnp.sum(e, axis=-1, keepdims=True)
        attn_ref[0] = a
        ctx = jnp.dot(a, mem, preferred_element_type=F32)
        proj = (jnp.dot(ctx, pwt_ref[...], preferred_element_type=F32)
                + jnp.dot(aob, pwb_ref[...], preferred_element_type=F32)
                + pb_ref[...])
        proj_ref[0] = proj
        xw_ref[0] = jnp.dot(proj, wih_ref[...], preferred_element_type=F32) + bih_ref[...]

    grid = (Nb, T // tb)
    n0 = lambda n, t: (n, 0, 0)
    nt = lambda n, t: (n, t, 0)
    full2 = lambda shape: pl.BlockSpec(shape, lambda n, t: (0, 0))
    out_shapes = (
        jax.ShapeDtypeStruct((Nb, T, TX), F32),
        jax.ShapeDtypeStruct((Nb, T, Eh), F32),
        jax.ShapeDtypeStruct((Nb, T, H3), F32),
    )
    return pl.pallas_call(
        body,
        grid=grid,
        in_specs=[
            pl.BlockSpec((1, TX, Eh), n0),
            pl.BlockSpec((1, tb, Eh), nt),
            full2((Eh, Eh)),
            full2((Eh, Eh)),
            full2((1, Eh)),
            full2((Eh, Eh)),
            full2((Eh, Eh)),
            full2((1, Eh)),
            full2((Eh, H3)),
            full2((1, H3)),
        ],
        out_specs=(
            pl.BlockSpec((1, tb, TX), nt),
            pl.BlockSpec((1, tb, Eh), nt),
            pl.BlockSpec((1, tb, H3), nt),
        ),
        out_shape=out_shapes,
        compiler_params=_cp(dimension_semantics=("arbitrary", "arbitrary"),
                            vmem_limit_bytes=100 * 1024 * 1024),
        name="attention_fused",
    )(memory, ao, W, U, v.reshape(1, Eh), pw[:Eh], pw[Eh:], pb.reshape(1, Eh),
      wih1T, bih1.reshape(1, H3))


def _conv_bank(y, wbig):
    C = wbig.shape[2]

    def body(y_ref, w_ref, bank_ref, s_ref, q_ref):
        x = y_ref[0]
        xp = jnp.concatenate(
            [jnp.zeros((3, N_MELS), F32), x, jnp.zeros((4, N_MELS), F32)], axis=0)
        acc = jnp.dot(xp[0:TY], w_ref[0], preferred_element_type=F32)
        for d in range(1, 8):
            acc += jnp.dot(xp[d:d + TY], w_ref[d], preferred_element_type=F32)
        bank_ref[0] = acc
        s_ref[0] = jnp.sum(acc, axis=0, keepdims=True)
        q_ref[0] = jnp.sum(acc * acc, axis=0, keepdims=True)

    return pl.pallas_call(
        body,
        grid=(N,),
        in_specs=[
            pl.BlockSpec((1, TY, N_MELS), lambda n: (n, 0, 0)),
            pl.BlockSpec((8, N_MELS, C), lambda n: (0, 0, 0)),
        ],
        out_specs=(
            pl.BlockSpec((1, TY, C), lambda n: (n, 0, 0)),
            pl.BlockSpec((1, 1, C), lambda n: (n, 0, 0)),
            pl.BlockSpec((1, 1, C), lambda n: (n, 0, 0)),
        ),
        out_shape=(
            jax.ShapeDtypeStruct((N, TY, C), F32),
            jax.ShapeDtypeStruct((N, 1, C), F32),
            jax.ShapeDtypeStruct((N, 1, C), F32),
        ),
        compiler_params=_cp(dimension_semantics=("arbitrary",),
                            vmem_limit_bytes=100 * 1024 * 1024),
        name="conv_bank",
    )(y, wbig)


def _bn_scale_shift(s_ref, q_ref, g_ref, be_ref):
    m = jnp.sum(s_ref[...], axis=0) / CNT
    var = jnp.sum(q_ref[...], axis=0) / CNT - m * m
    scale = g_ref[...] * jax.lax.rsqrt(var + 1e-5)
    shift = be_ref[...] - m * scale
    return scale, shift


def _conv_proj(xraw, ssum, ssq, g, be, w3, relu_in, pool_in, name):
    Cin = xraw.shape[2]
    Cout = w3.shape[2]

    def body(x_ref, s_in_ref, q_in_ref, g_ref, be_ref, w_ref,
             o_ref, s_ref, q_ref):
        scale, shift = _bn_scale_shift(s_in_ref, q_in_ref, g_ref, be_ref)
        a = x_ref[0] * scale + shift
        if relu_in:
            a = jnp.maximum(a, 0.0)
        if pool_in:
            an = jnp.concatenate([a[1:], jnp.zeros((1, Cin), F32)], axis=0)
            a = jnp.maximum(a, an)
        zp = jnp.zeros((1, Cin), F32)
        prv = jnp.concatenate([zp, a[:-1]], axis=0)
        nxt = jnp.concatenate([a[1:], zp], axis=0)
        acc = (jnp.dot(prv, w_ref[0], preferred_element_type=F32)
               + jnp.dot(a, w_ref[1], preferred_element_type=F32)
               + jnp.dot(nxt, w_ref[2], preferred_element_type=F32))
        o_ref[0] = acc
        s_ref[0] = jnp.sum(acc, axis=0, keepdims=True)
        q_ref[0] = jnp.sum(acc * acc, axis=0, keepdims=True)

    return pl.pallas_call(
        body,
        grid=(N,),
        in_specs=[
            pl.BlockSpec((1, TY, Cin), lambda n: (n, 0, 0)),
            pl.BlockSpec((N, 1, Cin), lambda n: (0, 0, 0)),
            pl.BlockSpec((N, 1, Cin), lambda n: (0, 0, 0)),
            pl.BlockSpec((1, Cin), lambda n: (0, 0)),
            pl.BlockSpec((1, Cin), lambda n: (0, 0)),
            pl.BlockSpec((3, Cin, Cout), lambda n: (0, 0, 0)),
        ],
        out_specs=(
            pl.BlockSpec((1, TY, Cout), lambda n: (n, 0, 0)),
            pl.BlockSpec((1, 1, Cout), lambda n: (n, 0, 0)),
            pl.BlockSpec((1, 1, Cout), lambda n: (n, 0, 0)),
        ),
        out_shape=(
            jax.ShapeDtypeStruct((N, TY, Cout), F32),
            jax.ShapeDtypeStruct((N, 1, Cout), F32),
            jax.ShapeDtypeStruct((N, 1, Cout), F32),
        ),
        compiler_params=_cp(dimension_semantics=("arbitrary",),
                            vmem_limit_bytes=100 * 1024 * 1024),
        name=name,
    )(xraw, ssum.reshape(N, 1, Cin), ssq.reshape(N, 1, Cin),
      g.reshape(1, Cin), be.reshape(1, Cin), w3)


def _highway(c2f, yf, ssum, ssq, g, be, wpre, bpre, hw_ws, wfT, bf, wbT, bb,
             row_block=512):
    M = c2f.shape[0]
    H3 = wfT.shape[1]

    def body(*refs):
        (c2_ref, y_ref, s_ref, q_ref, g_ref, be_ref, wp_ref, bp_ref) = refs[:8]
        hw_refs = refs[8:8 + 16]
        (wf_ref, bf_ref, wb_ref, bb_ref, xf_ref, xb_ref) = refs[24:]
        scale, shift = _bn_scale_shift(s_ref, q_ref, g_ref, be_ref)
        h0 = c2_ref[...] * scale + shift + y_ref[...]
        h = jnp.dot(h0, wp_ref[...], preferred_element_type=F32) + bp_ref[...]
        for i in range(4):
            w1_ref, b1_ref, w2_ref, b2_ref = hw_refs[4 * i:4 * i + 4]
            hh = jnp.maximum(
                jnp.dot(h, w1_ref[...], preferred_element_type=F32) + b1_ref[...], 0.0)
            tt = jax.nn.sigmoid(
                jnp.dot(h, w2_ref[...], preferred_element_type=F32) + b2_ref[...])
            h = hh * tt + h * (1.0 - tt)
        xf_ref[...] = jnp.dot(h, wf_ref[...], preferred_element_type=F32) + bf_ref[...]
        xb_ref[...] = jnp.dot(h, wb_ref[...], preferred_element_type=F32) + bb_ref[...]

    full = lambda shape: pl.BlockSpec(shape, lambda i: tuple(0 for _ in shape))
    rows = lambda c: pl.BlockSpec((row_block, c), lambda i: (i, 0))
    hw_args, hw_specs = [], []
    for (w1, b1, w2, b2) in hw_ws:
        hw_args += [w1, b1.reshape(1, HALF), w2, b2.reshape(1, HALF)]
        hw_specs += [full((HALF, HALF)), full((1, HALF)),
                     full((HALF, HALF)), full((1, HALF))]
    return pl.pallas_call(
        body,
        grid=(M // row_block,),
        in_specs=[
            rows(N_MELS), rows(N_MELS),
            full((N, 1, N_MELS)), full((N, 1, N_MELS)),
            full((1, N_MELS)), full((1, N_MELS)),
            full((N_MELS, HALF)), full((1, HALF)),
        ] + hw_specs + [
            full((HALF, H3)), full((1, H3)),
            full((HALF, H3)), full((1, H3)),
        ],
        out_specs=(
            pl.BlockSpec((row_block, H3), lambda i: (i, 0)),
            pl.BlockSpec((row_block, H3), lambda i: (i, 0)),
        ),
        out_shape=(
            jax.ShapeDtypeStruct((M, H3), F32),
            jax.ShapeDtypeStruct((M, H3), F32),
        ),
        compiler_params=_cp(dimension_semantics=("arbitrary",)),
        name="highway_stack",
    )(c2f, yf, ssum.reshape(N, 1, N_MELS), ssq.reshape(N, 1, N_MELS),
      g.reshape(1, N_MELS), be.reshape(1, N_MELS),
      wpre, bpre.reshape(1, HALF), *hw_args,
      wfT, bf.reshape(1, H3), wbT, bb.reshape(1, H3))


def _bigru(xwf, xwb, whhTf, bhhf, whhTb, bhhb, tb=16):
    Nb, T, H3 = xwf.shape
    H = H3 // 3
    nblk = T // tb

    def body(xf_ref, xb_ref, wf_ref, bf_ref, wb_ref, bb_ref,
             yf_ref, yb_ref, hf_ref, hb_ref):
        @pl.when(pl.program_id(0) == 0)
        def _():
            hf_ref[...] = jnp.zeros_like(hf_ref)
            hb_ref[...] = jnp.zeros_like(hb_ref)

        hf = hf_ref[...]
        hb = hb_ref[...]
        for j in range(tb):
            hf = _gru_cell(xf_ref[:, j, :], hf, wf_ref, bf_ref, H)
            yf_ref[:, j, :] = hf
            jj = tb - 1 - j
            hb = _gru_cell(xb_ref[:, jj, :], hb, wb_ref, bb_ref, H)
            yb_ref[:, jj, :] = hb
        hf_ref[...] = hf
        hb_ref[...] = hb

    fwd = lambda t: (0, t, 0)
    rev = lambda t: (0, nblk - 1 - t, 0)
    return pl.pallas_call(
        body,
        grid=(nblk,),
        in_specs=[
            pl.BlockSpec((Nb, tb, H3), fwd),
            pl.BlockSpec((Nb, tb, H3), rev),
            pl.BlockSpec((H, H3), lambda t: (0, 0)),
            pl.BlockSpec((1, H3), lambda t: (0, 0)),
            pl.BlockSpec((H, H3), lambda t: (0, 0)),
            pl.BlockSpec((1, H3), lambda t: (0, 0)),
        ],
        out_specs=(
            pl.BlockSpec((Nb, tb, H), fwd),
            pl.BlockSpec((Nb, tb, H), rev),
        ),
        out_shape=(
            jax.ShapeDtypeStruct((Nb, T, H), F32),
            jax.ShapeDtypeStruct((Nb, T, H), F32),
        ),
        scratch_shapes=[pltpu.VMEM((Nb, H), F32), pltpu.VMEM((Nb, H), F32)],
        compiler_params=_cp(dimension_semantics=("arbitrary",)),
        name="bigru",
    )(xwf, xwb, whhTf, bhhf, whhTb, bhhb)


def _fc2(f, b, w, bias, row_block=512):
    M = f.shape[0]
    Hh = f.shape[1]
    n_out = w.shape[1]

    def body(f_ref, b_ref, w_ref, bi_ref, o_ref):
        o = (jnp.dot(f_ref[...], w_ref[:Hh], preferred_element_type=F32)
             + jnp.dot(b_ref[...], w_ref[Hh:], preferred_element_type=F32)
             + bi_ref[...])
        o_ref[...] = jnp.transpose(o)

    return pl.pallas_call(
        body,
        grid=(M // row_block,),
        in_specs=[
            pl.BlockSpec((row_block, Hh), lambda i: (i, 0)),
            pl.BlockSpec((row_block, Hh), lambda i: (i, 0)),
            pl.BlockSpec((2 * Hh, n_out), lambda i: (0, 0)),
            pl.BlockSpec((1, n_out), lambda i: (0, 0)),
        ],
        out_specs=pl.BlockSpec((n_out, row_block), lambda i: (0, i)),
        out_shape=jax.ShapeDtypeStruct((n_out, M), F32),
        compiler_params=_cp(dimension_semantics=("arbitrary",)),
        name="fc2",
    )(f, b, w, bias.reshape(1, n_out))


def kernel(inputs, memory, params):
    p = params
    NT = N * TYR
    NTY = N * TY

    (wa_ihT, wa_hhT, g1_ihT, g1_hhT, g2_ihT, g2_hhT,
     bgf_ihT, bgf_hhT, bgb_ihT, bgb_hhT, wbig, g_cat, be_cat) = _prep(
        [p['attn_gru']['wih'], p['attn_gru']['whh'],
         p['gru1']['wih'], p['gru1']['whh'],
         p['gru2']['wih'], p['gru2']['whh'],
         p['bigru_f']['wih'], p['bigru_f']['whh'],
         p['bigru_b']['wih'], p['bigru_b']['whh']],
        [c['w'] for c in p['bank']],
        [c['g'] for c in p['bank']],
        [c['be'] for c in p['bank']])

    xw_a = _dense_chain(
        inputs.reshape(NT, N_MELS),
        [(p['prenet']['w1'], p['prenet']['b1'], "relu"),
         (p['prenet']['w2'], p['prenet']['b2'], "relu"),
         (wa_ihT, p['attn_gru']['bih'], "none")],
        row_block=256, name="prenet_xw")
    ao = _gru_scan(xw_a.reshape(N, TYR, 3 * E), wa_hhT,
                   p['attn_gru']['bhh'], name="attn_gru")

    attn, proj, xw1 = _attention(
        memory, ao, p['attn']['W'], p['attn']['U'], p['attn']['v'],
        p['proj']['w'], p['proj']['b'], g1_ihT, p['gru1']['bih'])

    g1 = _gru_scan(xw1, g1_hhT, p['gru1']['bhh'], res=proj, name="gru1")
    xw2 = _dense_chain(g1.reshape(NT, E),
                       [(g2_ihT, p['gru2']['bih'], "none")],
                       row_block=256, name="gru2_xw")
    g2 = _gru_scan(xw2.reshape(N, TYR, 3 * E), g2_hhT,
                   p['gru2']['bhh'], res=g1, name="gru2")

    mels = _dense_chain(g2.reshape(NT, E),
                        [(p['fc1']['w'], p['fc1']['b'], "none")],
                        row_block=256, name="fc1").reshape(N, TYR, N_MELS * R)
    y = mels.reshape(N, TY, N_MELS)

    bank, bs, bq = _conv_bank(y, wbig)
    c1, c1s, c1q = _conv_proj(bank, bs, bq, g_cat, be_cat, p['proj1']['w'],
                              relu_in=True, pool_in=True, name="proj1_conv")
    c2, c2s, c2q = _conv_proj(c1, c1s, c1q, p['proj1']['g'], p['proj1']['be'],
                              p['proj2']['w'],
                              relu_in=True, pool_in=False, name="proj2_conv")

    hw_ws = [(hw['w1'], hw['b1'], hw['w2'], hw['b2']) for hw in p['hw']]
    xwf, xwb = _highway(
        c2.reshape(NTY, N_MELS), y.reshape(NTY, N_MELS),
        c2s, c2q, p['proj2']['g'], p['proj2']['be'],
        p['pre_hw']['w'], p['pre_hw']['b'], hw_ws,
        bgf_ihT, p['bigru_f']['bih'], bgb_ihT, p['bigru_b']['bih'])

    f, bwd = _bigru(xwf.reshape(N, TY, 3 * HALF), xwb.reshape(N, TY, 3 * HALF),
                    bgf_hhT, p['bigru_f']['bhh'].reshape(1, 3 * HALF),
                    bgb_hhT, p['bigru_b']['bhh'].reshape(1, 3 * HALF))

    mags_t = _fc2(f.reshape(NTY, HALF), bwd.reshape(NTY, HALF),
                  p['fc2']['w'], p['fc2']['b'])
    mags = mags_t.reshape(1 + NFFT // 2, N, TY).transpose(1, 2, 0)
    return mels, mags, attn


# speedup vs baseline: 1.0709x; 1.0377x over previous
import jax
import jax.numpy as jnp
from jax.experimental import pallas as pl
from jax.experimental.pallas import tpu as pltpu

F32 = jnp.float32
N_MELS, R, E, NFFT, K = 80, 5, 256, 2048, 8
HALF = E // 2
N, TYR, TX = 16, 160, 128
TY = TYR * R
CNT = float(N * TY)


def _cp(**kw):
    return pltpu.CompilerParams(**kw)


def _prep(gru_ws, bank_ws, bank_gs, bank_bes):
    n_gru = len(gru_ws)

    def body(*refs):
        ins = refs[:n_gru + K * 3]
        outs = refs[n_gru + K * 3:]
        for i in range(n_gru):
            outs[i][...] = jnp.transpose(ins[i][...])
        wbig_ref = outs[n_gru]
        g_ref = outs[n_gru + 1]
        be_ref = outs[n_gru + 2]
        wbig_ref[...] = jnp.zeros_like(wbig_ref)
        for j in range(K):
            kk = j + 1
            lo = 3 - (kk - 1) // 2
            wbig_ref[lo:lo + kk, :, j * HALF:(j + 1) * HALF] = ins[n_gru + j][...]
            g_ref[:, j * HALF:(j + 1) * HALF] = ins[n_gru + K + j][...]
            be_ref[:, j * HALF:(j + 1) * HALF] = ins[n_gru + 2 * K + j][...]

    full = lambda shape: pl.BlockSpec(shape, lambda: tuple(0 for _ in shape))
    args = list(gru_ws) + list(bank_ws) + \
        [g.reshape(1, HALF) for g in bank_gs] + \
        [b.reshape(1, HALF) for b in bank_bes]
    in_specs = [full(a.shape) for a in args]
    out_shapes = [jax.ShapeDtypeStruct((w.shape[1], w.shape[0]), F32)
                  for w in gru_ws]
    out_shapes += [jax.ShapeDtypeStruct((8, N_MELS, K * HALF), F32),
                   jax.ShapeDtypeStruct((1, K * HALF), F32),
                   jax.ShapeDtypeStruct((1, K * HALF), F32)]
    out_specs = tuple(full(s.shape) for s in out_shapes)
    return pl.pallas_call(
        body,
        in_specs=in_specs,
        out_specs=out_specs,
        out_shape=tuple(out_shapes),
        compiler_params=_cp(vmem_limit_bytes=100 * 1024 * 1024),
        name="weight_prep",
    )(*args)


def _dense_chain(x, layers, row_block, name):
    M, K0 = x.shape
    grid = (M // row_block,)

    def body(*refs):
        x_ref = refs[0]
        o_ref = refs[-1]
        h = x_ref[...]
        for i in range(len(layers)):
            w_ref = refs[1 + 2 * i]
            b_ref = refs[2 + 2 * i]
            h = jnp.dot(h, w_ref[...], preferred_element_type=F32) + b_ref[...]
            if layers[i][2] == "relu":
                h = jnp.maximum(h, 0.0)
        o_ref[...] = h

    in_specs = [pl.BlockSpec((row_block, K0), lambda i: (i, 0))]
    args = [x]
    for (w, b, _act) in layers:
        kk, nn = w.shape
        in_specs.append(pl.BlockSpec((kk, nn), lambda i: (0, 0)))
        in_specs.append(pl.BlockSpec((1, nn), lambda i: (0, 0)))
        args.append(w)
        args.append(b.reshape(1, nn))
    n_out = layers[-1][0].shape[1]
    return pl.pallas_call(
        body,
        grid=grid,
        in_specs=in_specs,
        out_specs=pl.BlockSpec((row_block, n_out), lambda i: (i, 0)),
        out_shape=jax.ShapeDtypeStruct((M, n_out), F32),
        compiler_params=_cp(dimension_semantics=("arbitrary",)),
        name=name,
    )(*args)


def _gru_cell(xt, h, whhT_ref, bhh_ref, H):
    hw = jnp.dot(h, whhT_ref[...], preferred_element_type=F32) + bhh_ref[...]
    r = jax.nn.sigmoid(xt[:, :H] + hw[:, :H])
    z = jax.nn.sigmoid(xt[:, H:2 * H] + hw[:, H:2 * H])
    c = jnp.tanh(xt[:, 2 * H:] + r * hw[:, 2 * H:])
    return (1.0 - z) * c + z * h


def _gru_scan(xw, whhT, bhh, res=None, tb=16, name="gru_scan"):
    Nb, T, H3 = xw.shape
    H = H3 // 3
    has_res = res is not None

    def body(*refs):
        if has_res:
            xw_ref, whh_ref, bhh_ref, res_ref, ys_ref, h_ref = refs
        else:
            xw_ref, whh_ref, bhh_ref, ys_ref, h_ref = refs

        @pl.when(pl.program_id(0) == 0)
        def _():
            h_ref[...] = jnp.zeros_like(h_ref)

        h = h_ref[...]
        for j in range(tb):
            h = _gru_cell(xw_ref[:, j, :], h, whh_ref, bhh_ref, H)
            if has_res:
                ys_ref[:, j, :] = h + res_ref[:, j, :]
            else:
                ys_ref[:, j, :] = h
        h_ref[...] = h

    in_specs = [
        pl.BlockSpec((Nb, tb, H3), lambda t: (0, t, 0)),
        pl.BlockSpec((H, H3), lambda t: (0, 0)),
        pl.BlockSpec((1, H3), lambda t: (0, 0)),
    ]
    args = [xw, whhT, bhh.reshape(1, H3)]
    if has_res:
        in_specs.append(pl.BlockSpec((Nb, tb, H), lambda t: (0, t, 0)))
        args.append(res)
    return pl.pallas_call(
        body,
        grid=(T // tb,),
        in_specs=in_specs,
        out_specs=pl.BlockSpec((Nb, tb, H), lambda t: (0, t, 0)),
        out_shape=jax.ShapeDtypeStruct((Nb, T, H), F32),
        scratch_shapes=[pltpu.VMEM((Nb, H), F32)],
        compiler_params=_cp(dimension_semantics=("arbitrary",)),
        name=name,
    )(*args)


def _attention(memory, ao, W, U, v, pw, pb, wih1T, bih1, tb=80):
    Nb, T, Eh = ao.shape
    H3 = wih1T.shape[1]

    def body(mem_ref, ao_ref, w_ref, u_ref, v_ref, pwt_ref, pwb_ref, pb_ref,
             wih_ref, bih_ref, attn_ref, proj_ref, xw_ref):
        mem = mem_ref[0]
        wm = jnp.dot(mem, w_ref[...], preferred_element_type=F32)
        aob = ao_ref[0]
        uo = jnp.dot(aob, u_ref[...], preferred_element_type=F32)
        tnh = jnp.tanh(wm[None, :, :] + uo[:, None, :])
        scores = jnp.sum(tnh * v_ref[...][None, :, :], axis=-1)
        m = jnp.max(scores, axis=-1, keepdims=True)
        e = jnp.exp(scores - m)
        a = e / jnp.sum(e, axis=-1, keepdims=True)
        attn_ref[0] = a
        ctx = jnp.dot(a, mem, preferred_element_type=F32)
        proj = (jnp.dot(ctx, pwt_ref[...], preferred_element_type=F32)
                + jnp.dot(aob, pwb_ref[...], preferred_element_type=F32)
                + pb_ref[...])
        proj_ref[0] = proj
        xw_ref[0] = jnp.dot(proj, wih_ref[...], preferred_element_type=F32) + bih_ref[...]

    grid = (Nb, T // tb)
    n0 = lambda n, t: (n, 0, 0)
    nt = lambda n, t: (n, t, 0)
    full2 = lambda shape: pl.BlockSpec(shape, lambda n, t: (0, 0))
    out_shapes = (
        jax.ShapeDtypeStruct((Nb, T, TX), F32),
        jax.ShapeDtypeStruct((Nb, T, Eh), F32),
        jax.ShapeDtypeStruct((Nb, T, H3), F32),
    )
    return pl.pallas_call(
        body,
        grid=grid,
        in_specs=[
            pl.BlockSpec((1, TX, Eh), n0),
            pl.BlockSpec((1, tb, Eh), nt),
            full2((Eh, Eh)),
            full2((Eh, Eh)),
            full2((1, Eh)),
            full2((Eh, Eh)),
            full2((Eh, Eh)),
            full2((1, Eh)),
            full2((Eh, H3)),
            full2((1, H3)),
        ],
        out_specs=(
            pl.BlockSpec((1, tb, TX), nt),
            pl.BlockSpec((1, tb, Eh), nt),
            pl.BlockSpec((1, tb, H3), nt),
        ),
        out_shape=out_shapes,
        compiler_params=_cp(dimension_semantics=("arbitrary", "arbitrary"),
                            vmem_limit_bytes=100 * 1024 * 1024),
        name="attention_fused",
    )(memory, ao, W, U, v.reshape(1, Eh), pw[:Eh], pw[Eh:], pb.reshape(1, Eh),
      wih1T, bih1.reshape(1, H3))


def _conv_bank(y, wbig):
    C = wbig.shape[2]

    def body(y_ref, w_ref, bank_ref, s_ref, q_ref):
        x = y_ref[0]
        xp = jnp.concatenate(
            [jnp.zeros((3, N_MELS), F32), x, jnp.zeros((4, N_MELS), F32)], axis=0)
        acc = jnp.dot(xp[0:TY], w_ref[0], preferred_element_type=F32)
        for d in range(1, 8):
            acc += jnp.dot(xp[d:d + TY], w_ref[d], preferred_element_type=F32)
        bank_ref[0] = acc
        s_ref[0] = jnp.sum(acc, axis=0, keepdims=True)
        q_ref[0] = jnp.sum(acc * acc, axis=0, keepdims=True)

    return pl.pallas_call(
        body,
        grid=(N,),
        in_specs=[
            pl.BlockSpec((1, TY, N_MELS), lambda n: (n, 0, 0)),
            pl.BlockSpec((8, N_MELS, C), lambda n: (0, 0, 0)),
        ],
        out_specs=(
            pl.BlockSpec((1, TY, C), lambda n: (n, 0, 0)),
            pl.BlockSpec((1, 1, C), lambda n: (n, 0, 0)),
            pl.BlockSpec((1, 1, C), lambda n: (n, 0, 0)),
        ),
        out_shape=(
            jax.ShapeDtypeStruct((N, TY, C), F32),
            jax.ShapeDtypeStruct((N, 1, C), F32),
            jax.ShapeDtypeStruct((N, 1, C), F32),
        ),
        compiler_params=_cp(dimension_semantics=("arbitrary",),
                            vmem_limit_bytes=100 * 1024 * 1024),
        name="conv_bank",
    )(y, wbig)


def _bn_scale_shift(s_ref, q_ref, g_ref, be_ref):
    m = jnp.sum(s_ref[...], axis=0) / CNT
    var = jnp.sum(q_ref[...], axis=0) / CNT - m * m
    scale = g_ref[...] * jax.lax.rsqrt(var + 1e-5)
    shift = be_ref[...] - m * scale
    return scale, shift


def _conv_proj(xraw, ssum, ssq, g, be, w3, relu_in, pool_in, name):
    Cin = xraw.shape[2]
    Cout = w3.shape[2]

    def body(x_ref, s_in_ref, q_in_ref, g_ref, be_ref, w_ref,
             o_ref, s_ref, q_ref):
        scale, shift = _bn_scale_shift(s_in_ref, q_in_ref, g_ref, be_ref)
        a = x_ref[0] * scale + shift
        if relu_in:
            a = jnp.maximum(a, 0.0)
        if pool_in:
            an = jnp.concatenate([a[1:], jnp.zeros((1, Cin), F32)], axis=0)
            a = jnp.maximum(a, an)
        zp = jnp.zeros((1, Cin), F32)
        prv = jnp.concatenate([zp, a[:-1]], axis=0)
        nxt = jnp.concatenate([a[1:], zp], axis=0)
        acc = (jnp.dot(prv, w_ref[0], preferred_element_type=F32)
               + jnp.dot(a, w_ref[1], preferred_element_type=F32)
               + jnp.dot(nxt, w_ref[2], preferred_element_type=F32))
        o_ref[0] = acc
        s_ref[0] = jnp.sum(acc, axis=0, keepdims=True)
        q_ref[0] = jnp.sum(acc * acc, axis=0, keepdims=True)

    return pl.pallas_call(
        body,
        grid=(N,),
        in_specs=[
            pl.BlockSpec((1, TY, Cin), lambda n: (n, 0, 0)),
            pl.BlockSpec((N, 1, Cin), lambda n: (0, 0, 0)),
            pl.BlockSpec((N, 1, Cin), lambda n: (0, 0, 0)),
            pl.BlockSpec((1, Cin), lambda n: (0, 0)),
            pl.BlockSpec((1, Cin), lambda n: (0, 0)),
            pl.BlockSpec((3, Cin, Cout), lambda n: (0, 0, 0)),
        ],
        out_specs=(
            pl.BlockSpec((1, TY, Cout), lambda n: (n, 0, 0)),
            pl.BlockSpec((1, 1, Cout), lambda n: (n, 0, 0)),
            pl.BlockSpec((1, 1, Cout), lambda n: (n, 0, 0)),
        ),
        out_shape=(
            jax.ShapeDtypeStruct((N, TY, Cout), F32),
            jax.ShapeDtypeStruct((N, 1, Cout), F32),
            jax.ShapeDtypeStruct((N, 1, Cout), F32),
        ),
        compiler_params=_cp(dimension_semantics=("arbitrary",),
                            vmem_limit_bytes=100 * 1024 * 1024),
        name=name,
    )(xraw, ssum.reshape(N, 1, Cin), ssq.reshape(N, 1, Cin),
      g.reshape(1, Cin), be.reshape(1, Cin), w3)


def _highway(c2f, yf, ssum, ssq, g, be, wpre, bpre, hw_ws, wfT, bf, wbT, bb,
             row_block=512):
    M = c2f.shape[0]
    H3 = wfT.shape[1]

    def body(*refs):
        (c2_ref, y_ref, s_ref, q_ref, g_ref, be_ref, wp_ref, bp_ref) = refs[:8]
        hw_refs = refs[8:8 + 16]
        (wf_ref, bf_ref, wb_ref, bb_ref, xf_ref, xb_ref) = refs[24:]
        scale, shift = _bn_scale_shift(s_ref, q_ref, g_ref, be_ref)
        h0 = c2_ref[...] * scale + shift + y_ref[...]
        h = jnp.dot(h0, wp_ref[...], preferred_element_type=F32) + bp_ref[...]
        for i in range(4):
            w1_ref, b1_ref, w2_ref, b2_ref = hw_refs[4 * i:4 * i + 4]
            hh = jnp.maximum(
                jnp.dot(h, w1_ref[...], preferred_element_type=F32) + b1_ref[...], 0.0)
            tt = jax.nn.sigmoid(
                jnp.dot(h, w2_ref[...], preferred_element_type=F32) + b2_ref[...])
            h = hh * tt + h * (1.0 - tt)
        xf_ref[...] = jnp.dot(h, wf_ref[...], preferred_element_type=F32) + bf_ref[...]
        xb_ref[...] = jnp.dot(h, wb_ref[...], preferred_element_type=F32) + bb_ref[...]

    full = lambda shape: pl.BlockSpec(shape, lambda i: tuple(0 for _ in shape))
    rows = lambda c: pl.BlockSpec((row_block, c), lambda i: (i, 0))
    hw_args, hw_specs = [], []
    for (w1, b1, w2, b2) in hw_ws:
        hw_args += [w1, b1.reshape(1, HALF), w2, b2.reshape(1, HALF)]
        hw_specs += [full((HALF, HALF)), full((1, HALF)),
                     full((HALF, HALF)), full((1, HALF))]
    return pl.pallas_call(
        body,
        grid=(M // row_block,),
        in_specs=[
            rows(N_MELS), rows(N_MELS),
            full((N, 1, N_MELS)), full((N, 1, N_MELS)),
            full((1, N_MELS)), full((1, N_MELS)),
            full((N_MELS, HALF)), full((1, HALF)),
        ] + hw_specs + [
            full((HALF, H3)), full((1, H3)),
            full((HALF, H3)), full((1, H3)),
        ],
        out_specs=(
            pl.BlockSpec((row_block, H3), lambda i: (i, 0)),
            pl.BlockSpec((row_block, H3), lambda i: (i, 0)),
        ),
        out_shape=(
            jax.ShapeDtypeStruct((M, H3), F32),
            jax.ShapeDtypeStruct((M, H3), F32),
        ),
        compiler_params=_cp(dimension_semantics=("arbitrary",)),
        name="highway_stack",
    )(c2f, yf, ssum.reshape(N, 1, N_MELS), ssq.reshape(N, 1, N_MELS),
      g.reshape(1, N_MELS), be.reshape(1, N_MELS),
      wpre, bpre.reshape(1, HALF), *hw_args,
      wfT, bf.reshape(1, H3), wbT, bb.reshape(1, H3))


def _bigru(xwf, xwb, whhTf, bhhf, whhTb, bhhb, tb=16):
    Nb, T, H3 = xwf.shape
    H = H3 // 3
    nblk = T // tb

    def body(xf_ref, xb_ref, wf_ref, bf_ref, wb_ref, bb_ref,
             yf_ref, yb_ref, hf_ref, hb_ref):
        @pl.when(pl.program_id(0) == 0)
        def _():
            hf_ref[...] = jnp.zeros_like(hf_ref)
            hb_ref[...] = jnp.zeros_like(hb_ref)

        hf = hf_ref[...]
        hb = hb_ref[...]
        for j in range(tb):
            hf = _gru_cell(xf_ref[:, j, :], hf, wf_ref, bf_ref, H)
            yf_ref[:, j, :] = hf
            jj = tb - 1 - j
            hb = _gru_cell(xb_ref[:, jj, :], hb, wb_ref, bb_ref, H)
            yb_ref[:, jj, :] = hb
        hf_ref[...] = hf
        hb_ref[...] = hb

    fwd = lambda t: (0, t, 0)
    rev = lambda t: (0, nblk - 1 - t, 0)
    return pl.pallas_call(
        body,
        grid=(nblk,),
        in_specs=[
            pl.BlockSpec((Nb, tb, H3), fwd),
            pl.BlockSpec((Nb, tb, H3), rev),
            pl.BlockSpec((H, H3), lambda t: (0, 0)),
            pl.BlockSpec((1, H3), lambda t: (0, 0)),
            pl.BlockSpec((H, H3), lambda t: (0, 0)),
            pl.BlockSpec((1, H3), lambda t: (0, 0)),
        ],
        out_specs=(
            pl.BlockSpec((Nb, tb, H), fwd),
            pl.BlockSpec((Nb, tb, H), rev),
        ),
        out_shape=(
            jax.ShapeDtypeStruct((Nb, T, H), F32),
            jax.ShapeDtypeStruct((Nb, T, H), F32),
        ),
        scratch_shapes=[pltpu.VMEM((Nb, H), F32), pltpu.VMEM((Nb, H), F32)],
        compiler_params=_cp(dimension_semantics=("arbitrary",)),
        name="bigru",
    )(xwf, xwb, whhTf, bhhf, whhTb, bhhb)


def _fc2(f, b, w, bias, row_block=512):
    M = f.shape[0]
    Hh = f.shape[1]
    n_out = w.shape[1]

    def body(f_ref, b_ref, w_ref, bi_ref, o_ref):
        o = (jnp.dot(f_ref[...], w_ref[:Hh], preferred_element_type=F32)
             + jnp.dot(b_ref[...], w_ref[Hh:], preferred_element_type=F32)
             + bi_ref[...])
        o_ref[...] = jnp.transpose(o)

    return pl.pallas_call(
        body,
        grid=(M // row_block,),
        in_specs=[
            pl.BlockSpec((row_block, Hh), lambda i: (i, 0)),
            pl.BlockSpec((row_block, Hh), lambda i: (i, 0)),
            pl.BlockSpec((2 * Hh, n_out), lambda i: (0, 0)),
            pl.BlockSpec((1, n_out), lambda i: (0, 0)),
        ],
        out_specs=pl.BlockSpec((n_out, row_block), lambda i: (0, i)),
        out_shape=jax.ShapeDtypeStruct((n_out, M), F32),
        compiler_params=_cp(dimension_semantics=("arbitrary",)),
        name="fc2",
    )(f, b, w, bias.reshape(1, n_out))


def kernel(inputs, memory, params):
    p = params
    NT = N * TYR
    NTY = N * TY

    (wa_ihT, wa_hhT, g1_ihT, g1_hhT, g2_ihT, g2_hhT,
     bgf_ihT, bgf_hhT, bgb_ihT, bgb_hhT, wbig, g_cat, be_cat) = _prep(
        [p['attn_gru']['wih'], p['attn_gru']['whh'],
         p['gru1']['wih'], p['gru1']['whh'],
         p['gru2']['wih'], p['gru2']['whh'],
         p['bigru_f']['wih'], p['bigru_f']['whh'],
         p['bigru_b']['wih'], p['bigru_b']['whh']],
        [c['w'] for c in p['bank']],
        [c['g'] for c in p['bank']],
        [c['be'] for c in p['bank']])

    xw_a = _dense_chain(
        inputs.reshape(NT, N_MELS),
        [(p['prenet']['w1'], p['prenet']['b1'], "relu"),
         (p['prenet']['w2'], p['prenet']['b2'], "relu"),
         (wa_ihT, p['attn_gru']['bih'], "none")],
        row_block=256, name="prenet_xw")
    ao = _gru_scan(xw_a.reshape(N, TYR, 3 * E), wa_hhT,
                   p['attn_gru']['bhh'], name="attn_gru")

    attn, proj, xw1 = _attention(
        memory, ao, p['attn']['W'], p['attn']['U'], p['attn']['v'],
        p['proj']['w'], p['proj']['b'], g1_ihT, p['gru1']['bih'])

    g1 = _gru_scan(xw1, g1_hhT, p['gru1']['bhh'], res=proj, name="gru1")
    xw2 = _dense_chain(g1.reshape(NT, E),
                       [(g2_ihT, p['gru2']['bih'], "none")],
                       row_block=256, name="gru2_xw")
    g2 = _gru_scan(xw2.reshape(N, TYR, 3 * E), g2_hhT,
                   p['gru2']['bhh'], res=g1, name="gru2")

    mels = _dense_chain(g2.reshape(NT, E),
                        [(p['fc1']['w'], p['fc1']['b'], "none")],
                        row_block=256, name="fc1").reshape(N, TYR, N_MELS * R)
    y = mels.reshape(N, TY, N_MELS)

    bank, bs, bq = _conv_bank(y, wbig)
    c1, c1s, c1q = _conv_proj(bank, bs, bq, g_cat, be_cat, p['proj1']['w'],
                              relu_in=True, pool_in=True, name="proj1_conv")
    c2, c2s, c2q = _conv_proj(c1, c1s, c1q, p['proj1']['g'], p['proj1']['be'],
                              p['proj2']['w'],
                              relu_in=True, pool_in=False, name="proj2_conv")

    hw_ws = [(hw['w1'], hw['b1'], hw['w2'], hw['b2']) for hw in p['hw']]
    xwf, xwb = _highway(
        c2.reshape(NTY, N_MELS), y.reshape(NTY, N_MELS),
        c2s, c2q, p['proj2']['g'], p['proj2']['be'],
        p['pre_hw']['w'], p['pre_hw']['b'], hw_ws,
        bgf_ihT, p['bigru_f']['bih'], bgb_ihT, p['bigru_b']['bih'])

    f, bwd = _bigru(xwf.reshape(N, TY, 3 * HALF), xwb.reshape(N, TY, 3 * HALF),
                    bgf_hhT, p['bigru_f']['bhh'].reshape(1, 3 * HALF),
                    bgb_hhT, p['bigru_b']['bhh'].reshape(1, 3 * HALF))

    mags_t = _fc2(f.reshape(NTY, HALF), bwd.reshape(NTY, HALF),
                  p['fc2']['w'], p['fc2']['b'])
    mags = mags_t.reshape(1 + NFFT // 2, N, TY).transpose(1, 2, 0)
    return mels, mags, attn


# speedup vs baseline: 1.0877x; 1.0156x over previous
import jax
import jax.numpy as jnp
from jax.experimental import pallas as pl
from jax.experimental.pallas import tpu as pltpu

F32 = jnp.float32
N_MELS, R, E, NFFT, K = 80, 5, 256, 2048, 8
HALF = E // 2
N, TYR, TX = 16, 160, 128
TY = TYR * R
CNT = float(N * TY)


def _cp(**kw):
    return pltpu.CompilerParams(**kw)


def _prep(gru_ws, bank_ws, bank_gs, bank_bes):
    n_gru = len(gru_ws)

    def body(*refs):
        ins = refs[:n_gru + K * 3]
        outs = refs[n_gru + K * 3:]
        for i in range(n_gru):
            outs[i][...] = jnp.transpose(ins[i][...])
        wbig_ref = outs[n_gru]
        g_ref = outs[n_gru + 1]
        be_ref = outs[n_gru + 2]
        wbig_ref[...] = jnp.zeros_like(wbig_ref)
        for j in range(K):
            kk = j + 1
            lo = 3 - (kk - 1) // 2
            wbig_ref[lo:lo + kk, :, j * HALF:(j + 1) * HALF] = ins[n_gru + j][...]
            g_ref[:, j * HALF:(j + 1) * HALF] = ins[n_gru + K + j][...]
            be_ref[:, j * HALF:(j + 1) * HALF] = ins[n_gru + 2 * K + j][...]

    full = lambda shape: pl.BlockSpec(shape, lambda: tuple(0 for _ in shape))
    args = list(gru_ws) + list(bank_ws) + \
        [g.reshape(1, HALF) for g in bank_gs] + \
        [b.reshape(1, HALF) for b in bank_bes]
    in_specs = [full(a.shape) for a in args]
    out_shapes = [jax.ShapeDtypeStruct((w.shape[1], w.shape[0]), F32)
                  for w in gru_ws]
    out_shapes += [jax.ShapeDtypeStruct((8, N_MELS, K * HALF), F32),
                   jax.ShapeDtypeStruct((1, K * HALF), F32),
                   jax.ShapeDtypeStruct((1, K * HALF), F32)]
    out_specs = tuple(full(s.shape) for s in out_shapes)
    return pl.pallas_call(
        body,
        in_specs=in_specs,
        out_specs=out_specs,
        out_shape=tuple(out_shapes),
        compiler_params=_cp(vmem_limit_bytes=100 * 1024 * 1024),
        name="weight_prep",
    )(*args)


def _dense_chain(x, layers, row_block, name):
    M, K0 = x.shape
    grid = (M // row_block,)

    def body(*refs):
        x_ref = refs[0]
        o_ref = refs[-1]
        h = x_ref[...]
        for i in range(len(layers)):
            w_ref = refs[1 + 2 * i]
            b_ref = refs[2 + 2 * i]
            h = jnp.dot(h, w_ref[...], preferred_element_type=F32) + b_ref[...]
            if layers[i][2] == "relu":
                h = jnp.maximum(h, 0.0)
        o_ref[...] = h

    in_specs = [pl.BlockSpec((row_block, K0), lambda i: (i, 0))]
    args = [x]
    for (w, b, _act) in layers:
        kk, nn = w.shape
        in_specs.append(pl.BlockSpec((kk, nn), lambda i: (0, 0)))
        in_specs.append(pl.BlockSpec((1, nn), lambda i: (0, 0)))
        args.append(w)
        args.append(b.reshape(1, nn))
    n_out = layers[-1][0].shape[1]
    return pl.pallas_call(
        body,
        grid=grid,
        in_specs=in_specs,
        out_specs=pl.BlockSpec((row_block, n_out), lambda i: (i, 0)),
        out_shape=jax.ShapeDtypeStruct((M, n_out), F32),
        compiler_params=_cp(dimension_semantics=("arbitrary",)),
        name=name,
    )(*args)


def _gru_cell(xt, h, whhT_ref, bhh_ref, H):
    hw = jnp.dot(h, whhT_ref[...], preferred_element_type=F32) + bhh_ref[...]
    r = jax.nn.sigmoid(xt[:, :H] + hw[:, :H])
    z = jax.nn.sigmoid(xt[:, H:2 * H] + hw[:, H:2 * H])
    c = jnp.tanh(xt[:, 2 * H:] + r * hw[:, 2 * H:])
    return (1.0 - z) * c + z * h


def _gru_scan(xw, whhT, bhh, res=None, tb=16, name="gru_scan"):
    Nb, T, H3 = xw.shape
    H = H3 // 3
    has_res = res is not None

    def body(*refs):
        if has_res:
            xw_ref, whh_ref, bhh_ref, res_ref, ys_ref, h_ref = refs
        else:
            xw_ref, whh_ref, bhh_ref, ys_ref, h_ref = refs

        @pl.when(pl.program_id(0) == 0)
        def _():
            h_ref[...] = jnp.zeros_like(h_ref)

        h = h_ref[...]
        for j in range(tb):
            h = _gru_cell(xw_ref[:, j, :], h, whh_ref, bhh_ref, H)
            if has_res:
                ys_ref[:, j, :] = h + res_ref[:, j, :]
            else:
                ys_ref[:, j, :] = h
        h_ref[...] = h

    in_specs = [
        pl.BlockSpec((Nb, tb, H3), lambda t: (0, t, 0)),
        pl.BlockSpec((H, H3), lambda t: (0, 0)),
        pl.BlockSpec((1, H3), lambda t: (0, 0)),
    ]
    args = [xw, whhT, bhh.reshape(1, H3)]
    if has_res:
        in_specs.append(pl.BlockSpec((Nb, tb, H), lambda t: (0, t, 0)))
        args.append(res)
    return pl.pallas_call(
        body,
        grid=(T // tb,),
        in_specs=in_specs,
        out_specs=pl.BlockSpec((Nb, tb, H), lambda t: (0, t, 0)),
        out_shape=jax.ShapeDtypeStruct((Nb, T, H), F32),
        scratch_shapes=[pltpu.VMEM((Nb, H), F32)],
        compiler_params=_cp(dimension_semantics=("arbitrary",)),
        name=name,
    )(*args)


def _attention(memory, ao, W, U, v, pw, pb, wih1T, bih1, tb=160):
    Nb, T, Eh = ao.shape
    H3 = wih1T.shape[1]

    def body(mem_ref, ao_ref, w_ref, u_ref, v_ref, pwt_ref, pwb_ref, pb_ref,
             wih_ref, bih_ref, attn_ref, proj_ref, xw_ref):
        mem = mem_ref[0]
        wm = jnp.dot(mem, w_ref[...], preferred_element_type=F32)
        aob = ao_ref[0]
        uo = jnp.dot(aob, u_ref[...], preferred_element_type=F32)
        tnh = jnp.tanh(wm[None, :, :] + uo[:, None, :])
        scores = jnp.sum(tnh * v_ref[...][None, :, :], axis=-1)
        m = jnp.max(scores, axis=-1, keepdims=True)
        e = jnp.exp(scores - m)
        a = e / jnp.sum(e, axis=-1, keepdims=True)
        attn_ref[0] = a
        ctx = jnp.dot(a, mem, preferred_element_type=F32)
        proj = (jnp.dot(ctx, pwt_ref[...], preferred_element_type=F32)
                + jnp.dot(aob, pwb_ref[...], preferred_element_type=F32)
                + pb_ref[...])
        proj_ref[0] = proj
        xw_ref[0] = jnp.dot(proj, wih_ref[...], preferred_element_type=F32) + bih_ref[...]

    grid = (Nb, T // tb)
    n0 = lambda n, t: (n, 0, 0)
    nt = lambda n, t: (n, t, 0)
    full2 = lambda shape: pl.BlockSpec(shape, lambda n, t: (0, 0))
    out_shapes = (
        jax.ShapeDtypeStruct((Nb, T, TX), F32),
        jax.ShapeDtypeStruct((Nb, T, Eh), F32),
        jax.ShapeDtypeStruct((Nb, T, H3), F32),
    )
    return pl.pallas_call(
        body,
        grid=grid,
        in_specs=[
            pl.BlockSpec((1, TX, Eh), n0),
            pl.BlockSpec((1, tb, Eh), nt),
            full2((Eh, Eh)),
            full2((Eh, Eh)),
            full2((1, Eh)),
            full2((Eh, Eh)),
            full2((Eh, Eh)),
            full2((1, Eh)),
            full2((Eh, H3)),
            full2((1, H3)),
        ],
        out_specs=(
            pl.BlockSpec((1, tb, TX), nt),
            pl.BlockSpec((1, tb, Eh), nt),
            pl.BlockSpec((1, tb, H3), nt),
        ),
        out_shape=out_shapes,
        compiler_params=_cp(dimension_semantics=("arbitrary", "arbitrary"),
                            vmem_limit_bytes=100 * 1024 * 1024),
        name="attention_fused",
    )(memory, ao, W, U, v.reshape(1, Eh), pw[:Eh], pw[Eh:], pb.reshape(1, Eh),
      wih1T, bih1.reshape(1, H3))


def _conv_bank(y, wbig):
    C = wbig.shape[2]

    def body(y_ref, w_ref, bank_ref, s_ref, q_ref):
        x = y_ref[0]
        xp = jnp.concatenate(
            [jnp.zeros((3, N_MELS), F32), x, jnp.zeros((4, N_MELS), F32)], axis=0)
        acc = jnp.dot(xp[0:TY], w_ref[0], preferred_element_type=F32)
        for d in range(1, 8):
            acc += jnp.dot(xp[d:d + TY], w_ref[d], preferred_element_type=F32)
        bank_ref[0] = acc
        s_ref[0] = jnp.sum(acc, axis=0, keepdims=True)
        q_ref[0] = jnp.sum(acc * acc, axis=0, keepdims=True)

    return pl.pallas_call(
        body,
        grid=(N,),
        in_specs=[
            pl.BlockSpec((1, TY, N_MELS), lambda n: (n, 0, 0)),
            pl.BlockSpec((8, N_MELS, C), lambda n: (0, 0, 0)),
        ],
        out_specs=(
            pl.BlockSpec((1, TY, C), lambda n: (n, 0, 0)),
            pl.BlockSpec((1, 1, C), lambda n: (n, 0, 0)),
            pl.BlockSpec((1, 1, C), lambda n: (n, 0, 0)),
        ),
        out_shape=(
            jax.ShapeDtypeStruct((N, TY, C), F32),
            jax.ShapeDtypeStruct((N, 1, C), F32),
            jax.ShapeDtypeStruct((N, 1, C), F32),
        ),
        compiler_params=_cp(dimension_semantics=("arbitrary",),
                            vmem_limit_bytes=100 * 1024 * 1024),
        name="conv_bank",
    )(y, wbig)


def _bn_scale_shift(s_ref, q_ref, g_ref, be_ref):
    m = jnp.sum(s_ref[...], axis=0) / CNT
    var = jnp.sum(q_ref[...], axis=0) / CNT - m * m
    scale = g_ref[...] * jax.lax.rsqrt(var + 1e-5)
    shift = be_ref[...] - m * scale
    return scale, shift


def _conv_proj(xraw, ssum, ssq, g, be, w3, relu_in, pool_in, name):
    Cin = xraw.shape[2]
    Cout = w3.shape[2]

    def body(x_ref, s_in_ref, q_in_ref, g_ref, be_ref, w_ref,
             o_ref, s_ref, q_ref):
        scale, shift = _bn_scale_shift(s_in_ref, q_in_ref, g_ref, be_ref)
        a = x_ref[0] * scale + shift
        if relu_in:
            a = jnp.maximum(a, 0.0)
        if pool_in:
            an = jnp.concatenate([a[1:], jnp.zeros((1, Cin), F32)], axis=0)
            a = jnp.maximum(a, an)
        zp = jnp.zeros((1, Cin), F32)
        prv = jnp.concatenate([zp, a[:-1]], axis=0)
        nxt = jnp.concatenate([a[1:], zp], axis=0)
        acc = (jnp.dot(prv, w_ref[0], preferred_element_type=F32)
               + jnp.dot(a, w_ref[1], preferred_element_type=F32)
               + jnp.dot(nxt, w_ref[2], preferred_element_type=F32))
        o_ref[0] = acc
        s_ref[0] = jnp.sum(acc, axis=0, keepdims=True)
        q_ref[0] = jnp.sum(acc * acc, axis=0, keepdims=True)

    return pl.pallas_call(
        body,
        grid=(N,),
        in_specs=[
            pl.BlockSpec((1, TY, Cin), lambda n: (n, 0, 0)),
            pl.BlockSpec((N, 1, Cin), lambda n: (0, 0, 0)),
            pl.BlockSpec((N, 1, Cin), lambda n: (0, 0, 0)),
            pl.BlockSpec((1, Cin), lambda n: (0, 0)),
            pl.BlockSpec((1, Cin), lambda n: (0, 0)),
            pl.BlockSpec((3, Cin, Cout), lambda n: (0, 0, 0)),
        ],
        out_specs=(
            pl.BlockSpec((1, TY, Cout), lambda n: (n, 0, 0)),
            pl.BlockSpec((1, 1, Cout), lambda n: (n, 0, 0)),
            pl.BlockSpec((1, 1, Cout), lambda n: (n, 0, 0)),
        ),
        out_shape=(
            jax.ShapeDtypeStruct((N, TY, Cout), F32),
            jax.ShapeDtypeStruct((N, 1, Cout), F32),
            jax.ShapeDtypeStruct((N, 1, Cout), F32),
        ),
        compiler_params=_cp(dimension_semantics=("arbitrary",),
                            vmem_limit_bytes=100 * 1024 * 1024),
        name=name,
    )(xraw, ssum.reshape(N, 1, Cin), ssq.reshape(N, 1, Cin),
      g.reshape(1, Cin), be.reshape(1, Cin), w3)


def _highway(c2f, yf, ssum, ssq, g, be, wpre, bpre, hw_ws, wfT, bf, wbT, bb,
             row_block=512):
    M = c2f.shape[0]
    H3 = wfT.shape[1]

    def body(*refs):
        (c2_ref, y_ref, s_ref, q_ref, g_ref, be_ref, wp_ref, bp_ref) = refs[:8]
        hw_refs = refs[8:8 + 16]
        (wf_ref, bf_ref, wb_ref, bb_ref, xf_ref, xb_ref) = refs[24:]
        scale, shift = _bn_scale_shift(s_ref, q_ref, g_ref, be_ref)
        h0 = c2_ref[...] * scale + shift + y_ref[...]
        h = jnp.dot(h0, wp_ref[...], preferred_element_type=F32) + bp_ref[...]
        for i in range(4):
            w1_ref, b1_ref, w2_ref, b2_ref = hw_refs[4 * i:4 * i + 4]
            hh = jnp.maximum(
                jnp.dot(h, w1_ref[...], preferred_element_type=F32) + b1_ref[...], 0.0)
            tt = jax.nn.sigmoid(
                jnp.dot(h, w2_ref[...], preferred_element_type=F32) + b2_ref[...])
            h = hh * tt + h * (1.0 - tt)
        xf_ref[...] = jnp.dot(h, wf_ref[...], preferred_element_type=F32) + bf_ref[...]
        xb_ref[...] = jnp.dot(h, wb_ref[...], preferred_element_type=F32) + bb_ref[...]

    full = lambda shape: pl.BlockSpec(shape, lambda i: tuple(0 for _ in shape))
    rows = lambda c: pl.BlockSpec((row_block, c), lambda i: (i, 0))
    hw_args, hw_specs = [], []
    for (w1, b1, w2, b2) in hw_ws:
        hw_args += [w1, b1.reshape(1, HALF), w2, b2.reshape(1, HALF)]
        hw_specs += [full((HALF, HALF)), full((1, HALF)),
                     full((HALF, HALF)), full((1, HALF))]
    return pl.pallas_call(
        body,
        grid=(M // row_block,),
        in_specs=[
            rows(N_MELS), rows(N_MELS),
            full((N, 1, N_MELS)), full((N, 1, N_MELS)),
            full((1, N_MELS)), full((1, N_MELS)),
            full((N_MELS, HALF)), full((1, HALF)),
        ] + hw_specs + [
            full((HALF, H3)), full((1, H3)),
            full((HALF, H3)), full((1, H3)),
        ],
        out_specs=(
            pl.BlockSpec((row_block, H3), lambda i: (i, 0)),
            pl.BlockSpec((row_block, H3), lambda i: (i, 0)),
        ),
        out_shape=(
            jax.ShapeDtypeStruct((M, H3), F32),
            jax.ShapeDtypeStruct((M, H3), F32),
        ),
        compiler_params=_cp(dimension_semantics=("arbitrary",)),
        name="highway_stack",
    )(c2f, yf, ssum.reshape(N, 1, N_MELS), ssq.reshape(N, 1, N_MELS),
      g.reshape(1, N_MELS), be.reshape(1, N_MELS),
      wpre, bpre.reshape(1, HALF), *hw_args,
      wfT, bf.reshape(1, H3), wbT, bb.reshape(1, H3))


def _bigru(xwf, xwb, whhTf, bhhf, whhTb, bhhb, tb=16):
    Nb, T, H3 = xwf.shape
    H = H3 // 3
    nblk = T // tb

    def body(xf_ref, xb_ref, wf_ref, bf_ref, wb_ref, bb_ref,
             yf_ref, yb_ref, hf_ref, hb_ref):
        @pl.when(pl.program_id(0) == 0)
        def _():
            hf_ref[...] = jnp.zeros_like(hf_ref)
            hb_ref[...] = jnp.zeros_like(hb_ref)

        hf = hf_ref[...]
        hb = hb_ref[...]
        for j in range(tb):
            hf = _gru_cell(xf_ref[:, j, :], hf, wf_ref, bf_ref, H)
            yf_ref[:, j, :] = hf
            jj = tb - 1 - j
            hb = _gru_cell(xb_ref[:, jj, :], hb, wb_ref, bb_ref, H)
            yb_ref[:, jj, :] = hb
        hf_ref[...] = hf
        hb_ref[...] = hb

    fwd = lambda t: (0, t, 0)
    rev = lambda t: (0, nblk - 1 - t, 0)
    return pl.pallas_call(
        body,
        grid=(nblk,),
        in_specs=[
            pl.BlockSpec((Nb, tb, H3), fwd),
            pl.BlockSpec((Nb, tb, H3), rev),
            pl.BlockSpec((H, H3), lambda t: (0, 0)),
            pl.BlockSpec((1, H3), lambda t: (0, 0)),
            pl.BlockSpec((H, H3), lambda t: (0, 0)),
            pl.BlockSpec((1, H3), lambda t: (0, 0)),
        ],
        out_specs=(
            pl.BlockSpec((Nb, tb, H), fwd),
            pl.BlockSpec((Nb, tb, H), rev),
        ),
        out_shape=(
            jax.ShapeDtypeStruct((Nb, T, H), F32),
            jax.ShapeDtypeStruct((Nb, T, H), F32),
        ),
        scratch_shapes=[pltpu.VMEM((Nb, H), F32), pltpu.VMEM((Nb, H), F32)],
        compiler_params=_cp(dimension_semantics=("arbitrary",)),
        name="bigru",
    )(xwf, xwb, whhTf, bhhf, whhTb, bhhb)


def _fc2(f, b, w, bias, row_block=512):
    M = f.shape[0]
    Hh = f.shape[1]
    n_out = w.shape[1]

    def body(f_ref, b_ref, w_ref, bi_ref, o_ref):
        o = (jnp.dot(f_ref[...], w_ref[:Hh], preferred_element_type=F32)
             + jnp.dot(b_ref[...], w_ref[Hh:], preferred_element_type=F32)
             + bi_ref[...])
        o_ref[...] = jnp.transpose(o)

    return pl.pallas_call(
        body,
        grid=(M // row_block,),
        in_specs=[
            pl.BlockSpec((row_block, Hh), lambda i: (i, 0)),
            pl.BlockSpec((row_block, Hh), lambda i: (i, 0)),
            pl.BlockSpec((2 * Hh, n_out), lambda i: (0, 0)),
            pl.BlockSpec((1, n_out), lambda i: (0, 0)),
        ],
        out_specs=pl.BlockSpec((n_out, row_block), lambda i: (0, i)),
        out_shape=jax.ShapeDtypeStruct((n_out, M), F32),
        compiler_params=_cp(dimension_semantics=("arbitrary",)),
        name="fc2",
    )(f, b, w, bias.reshape(1, n_out))


def kernel(inputs, memory, params):
    p = params
    NT = N * TYR
    NTY = N * TY

    (wa_ihT, wa_hhT, g1_ihT, g1_hhT, g2_ihT, g2_hhT,
     bgf_ihT, bgf_hhT, bgb_ihT, bgb_hhT, wbig, g_cat, be_cat) = _prep(
        [p['attn_gru']['wih'], p['attn_gru']['whh'],
         p['gru1']['wih'], p['gru1']['whh'],
         p['gru2']['wih'], p['gru2']['whh'],
         p['bigru_f']['wih'], p['bigru_f']['whh'],
         p['bigru_b']['wih'], p['bigru_b']['whh']],
        [c['w'] for c in p['bank']],
        [c['g'] for c in p['bank']],
        [c['be'] for c in p['bank']])

    xw_a = _dense_chain(
        inputs.reshape(NT, N_MELS),
        [(p['prenet']['w1'], p['prenet']['b1'], "relu"),
         (p['prenet']['w2'], p['prenet']['b2'], "relu"),
         (wa_ihT, p['attn_gru']['bih'], "none")],
        row_block=256, name="prenet_xw")
    ao = _gru_scan(xw_a.reshape(N, TYR, 3 * E), wa_hhT,
                   p['attn_gru']['bhh'], name="attn_gru")

    attn, proj, xw1 = _attention(
        memory, ao, p['attn']['W'], p['attn']['U'], p['attn']['v'],
        p['proj']['w'], p['proj']['b'], g1_ihT, p['gru1']['bih'])

    g1 = _gru_scan(xw1, g1_hhT, p['gru1']['bhh'], res=proj, name="gru1")
    xw2 = _dense_chain(g1.reshape(NT, E),
                       [(g2_ihT, p['gru2']['bih'], "none")],
                       row_block=256, name="gru2_xw")
    g2 = _gru_scan(xw2.reshape(N, TYR, 3 * E), g2_hhT,
                   p['gru2']['bhh'], res=g1, name="gru2")

    mels = _dense_chain(g2.reshape(NT, E),
                        [(p['fc1']['w'], p['fc1']['b'], "none")],
                        row_block=256, name="fc1").reshape(N, TYR, N_MELS * R)
    y = mels.reshape(N, TY, N_MELS)

    bank, bs, bq = _conv_bank(y, wbig)
    c1, c1s, c1q = _conv_proj(bank, bs, bq, g_cat, be_cat, p['proj1']['w'],
                              relu_in=True, pool_in=True, name="proj1_conv")
    c2, c2s, c2q = _conv_proj(c1, c1s, c1q, p['proj1']['g'], p['proj1']['be'],
                              p['proj2']['w'],
                              relu_in=True, pool_in=False, name="proj2_conv")

    hw_ws = [(hw['w1'], hw['b1'], hw['w2'], hw['b2']) for hw in p['hw']]
    xwf, xwb = _highway(
        c2.reshape(NTY, N_MELS), y.reshape(NTY, N_MELS),
        c2s, c2q, p['proj2']['g'], p['proj2']['be'],
        p['pre_hw']['w'], p['pre_hw']['b'], hw_ws,
        bgf_ihT, p['bigru_f']['bih'], bgb_ihT, p['bigru_b']['bih'])

    f, bwd = _bigru(xwf.reshape(N, TY, 3 * HALF), xwb.reshape(N, TY, 3 * HALF),
                    bgf_hhT, p['bigru_f']['bhh'].reshape(1, 3 * HALF),
                    bgb_hhT, p['bigru_b']['bhh'].reshape(1, 3 * HALF))

    mags_t = _fc2(f.reshape(NTY, HALF), bwd.reshape(NTY, HALF),
                  p['fc2']['w'], p['fc2']['b'])
    mags = mags_t.reshape(1 + NFFT // 2, N, TY).transpose(1, 2, 0)
    return mels, mags, attn
